# Initial kernel scaffold; baseline (speedup 1.0000x reference)
import jax
import jax.numpy as jnp
from jax.experimental import pallas as pl


def kernel(user_emb, item_emb, W_gc, b_gc, W_bi, b_bi, edge_val, edge_row, edge_col, user_id, item_id, neg_item_id):
    raise NotImplementedError("write your pallas kernel here")


# speedup vs baseline: 2.5500x; 2.5500x over previous
import functools

import jax
import jax.numpy as jnp
from jax import lax
from jax.experimental import pallas as pl
from jax.experimental.pallas import tpu as pltpu
from jax.experimental.pallas import tpu_sc as plsc

USER_NUM = 25000
ITEM_NUM = 25000
N = USER_NUM + ITEM_NUM
D = 64
H = 32
L = 3
NNZ = 800000
B = 4096
DECAY = 1e-4

NC = 2
NS = 16
NP = 50048
E = 128
CHUNKS = 391
EPT = E * CHUNKS
PADN = EPT * NS
RPT = NP // NS
BPW = B // (NC * NS)


def _prop_body(ego_lo, ego_hi, col_h, row_h, val_h, zeros_h,
               side_lo, side_hi, acc, colv, rowv, valv, rows, sem):
  c = lax.axis_index("c")
  s = lax.axis_index("s")

  def half(ego_h, side_h):
    pltpu.sync_copy(zeros_h.at[pl.ds(s * RPT, RPT)],
                    acc.at[pl.ds(s * RPT, RPT)])
    plsc.subcore_barrier()

    def chunk(j, carry):
      base = s * EPT + j * E
      pltpu.sync_copy(col_h.at[pl.ds(base, E)], colv)
      pltpu.sync_copy(row_h.at[pl.ds(base, E)], rowv)
      pltpu.sync_copy(val_h.at[pl.ds(base, E)], valv)
      pltpu.async_copy(ego_h.at[colv], rows, sem).wait()

      def group(g, carry):
        vals = valv[pl.ds(g * 16, 16)]
        for j in range(16):
          e = g * 16 + j
          v = vals[j]
          for h in range(H // 16):
            rows[e, pl.ds(h * 16, 16)] = rows[e, pl.ds(h * 16, 16)] * v
        return carry

      lax.fori_loop(0, E // 16, group, 0)
      pltpu.sync_copy(rows, acc.at[rowv], add=True)
      return carry

    lax.fori_loop(0, CHUNKS, chunk, 0)
    plsc.subcore_barrier()
    pltpu.sync_copy(acc.at[pl.ds(s * RPT, RPT)],
                    side_h.at[pl.ds(s * RPT, RPT)])

  pl.when(c == 0)(lambda: half(ego_lo, side_lo))
  pl.when(c == 1)(lambda: half(ego_hi, side_hi))


_prop = pl.kernel(
    _prop_body,
    out_type=[jax.ShapeDtypeStruct((NP, H), jnp.float32),
              jax.ShapeDtypeStruct((NP, H), jnp.float32)],
    mesh=plsc.VectorSubcoreMesh(core_axis_name="c", subcore_axis_name="s"),
    compiler_params=pltpu.CompilerParams(use_tc_tiling_on_sc=False),
    scratch_types=[
        pltpu.VMEM_SHARED((NP, H), jnp.float32),
        pltpu.VMEM((E,), jnp.int32),
        pltpu.VMEM((E,), jnp.int32),
        pltpu.VMEM((E,), jnp.float32),
        pltpu.VMEM((E, H), jnp.float32),
        pltpu.SemaphoreType.DMA,
    ],
)


def _dense_body(slo, shi, elo, ehi, wg, bg, wb, bb, nlo, nhi, norm):
  wg_t = wg[0:32, :]
  wg_b = wg[32:64, :]
  wb_t = wb[0:32, :]
  wb_b = wb[32:64, :]
  dot = functools.partial(lax.dot_general,
                          dimension_numbers=(((1,), (0,)), ((), ())),
                          preferred_element_type=jnp.float32,
                          precision=lax.Precision.HIGHEST)
  sum_e = dot(slo[...], wg_t) + dot(shi[...], wg_b) + bg[0:1, :]
  bi = dot(elo[...] * slo[...], wb_t) + dot(ehi[...] * shi[...], wb_b) + bb[0:1, :]
  t = sum_e + bi
  x = jnp.where(t >= 0, t, 0.2 * t)
  nlo[...] = x[:, 0:32]
  nhi[...] = x[:, 32:64]
  nrm = jnp.maximum(jnp.sqrt(jnp.sum(x * x, axis=1, keepdims=True)), 1e-12)
  norm[...] = x / nrm


def _dense(slo, shi, elo, ehi, wg, bg, wb, bb):
  BR = 3128
  grid = NP // BR
  row_spec = pl.BlockSpec((BR, H), lambda i: (i, 0))
  w_spec = pl.BlockSpec((D, D), lambda i: (0, 0))
  b_spec = pl.BlockSpec((8, D), lambda i: (0, 0))
  return pl.pallas_call(
      _dense_body,
      grid=(grid,),
      in_specs=[row_spec, row_spec, row_spec, row_spec,
                w_spec, b_spec, w_spec, b_spec],
      out_specs=[row_spec, row_spec, pl.BlockSpec((BR, D), lambda i: (i, 0))],
      out_shape=[jax.ShapeDtypeStruct((NP, H), jnp.float32),
                 jax.ShapeDtypeStruct((NP, H), jnp.float32),
                 jax.ShapeDtypeStruct((NP, D), jnp.float32)],
  )(slo, shi, elo, ehi, wg, bg, wb, bb)


def _gather_body(t0, t1, t2, t3, ru, rp, rn, outu, outp, outn,
                 idxv, gbuf, sem):
  c = lax.axis_index("c")
  s = lax.axis_index("s")
  w = s * NC + c
  tables = [t0, t1, t2, t3]
  for ids_h, out_h in ((ru, outu), (rp, outp), (rn, outn)):
    pltpu.sync_copy(ids_h.at[pl.ds(w * BPW, BPW)], idxv)
    for t in range(4):
      pltpu.async_copy(tables[t].at[idxv], gbuf, sem).wait()
      pltpu.sync_copy(gbuf, out_h.at[pl.ds(w * BPW, BPW), t])


_gather = pl.kernel(
    _gather_body,
    out_type=[jax.ShapeDtypeStruct((B, 4, D), jnp.float32)] * 3,
    mesh=plsc.VectorSubcoreMesh(core_axis_name="c", subcore_axis_name="s"),
    compiler_params=pltpu.CompilerParams(use_tc_tiling_on_sc=False),
    scratch_types=[
        pltpu.VMEM((BPW,), jnp.int32),
        pltpu.VMEM((BPW, D), jnp.float32),
        pltpu.SemaphoreType.DMA,
    ],
)


def _loss_body(u, p, n, loss, bpr, reg):
  ps = jnp.sum(u[...] * p[...], axis=1)
  ns = jnp.sum(u[...] * n[...], axis=1)
  d = ps - ns
  ls = jnp.minimum(d, 0.0) - jnp.log1p(jnp.exp(-jnp.abs(d)))
  bpr_v = -jnp.sum(ls) / B
  u0 = u[:, 0:D]
  p0 = p[:, 0:D]
  n0 = n[:, 0:D]
  reg_v = DECAY * (jnp.sum(u0 * u0) + jnp.sum(p0 * p0) + jnp.sum(n0 * n0)) \
      / 2.0 / B
  loss[0, 0] = bpr_v + reg_v
  bpr[0, 0] = bpr_v
  reg[0, 0] = reg_v


def _loss(u, p, n):
  spec = pl.BlockSpec((B, 4 * D), lambda: (0, 0))
  return pl.pallas_call(
      _loss_body,
      in_specs=[spec, spec, spec],
      out_specs=[pl.BlockSpec(memory_space=pltpu.SMEM)] * 3,
      out_shape=[jax.ShapeDtypeStruct((1, 1), jnp.float32)] * 3,
  )(u, p, n)


def kernel(user_emb, item_emb, W_gc, b_gc, W_bi, b_bi,
           edge_val, edge_row, edge_col, user_id, item_id, neg_item_id):
  ego0 = jnp.concatenate([user_emb, item_emb,
                          jnp.zeros((NP - N, D), jnp.float32)], axis=0)
  elo = ego0[:, :H]
  ehi = ego0[:, H:]

  pad = PADN - NNZ
  pad_rows = (jnp.arange(pad, dtype=jnp.int32) * 64) % N
  col_p = jnp.concatenate([edge_col.astype(jnp.int32), pad_rows])
  row_p = jnp.concatenate([edge_row.astype(jnp.int32), pad_rows])
  val_p = jnp.concatenate([edge_val, jnp.zeros((pad,), jnp.float32)])
  zeros_h = jnp.zeros((NP, H), jnp.float32)

  tables = [ego0]
  for k in range(L):
    slo, shi = _prop(elo, ehi, col_p, row_p, val_p, zeros_h)
    bg = jnp.broadcast_to(b_gc[k], (8, D))
    bb = jnp.broadcast_to(b_bi[k], (8, D))
    elo, ehi, nrm = _dense(slo, shi, elo, ehi, W_gc[k], bg, W_bi[k], bb)
    tables.append(nrm)

  ru = user_id.astype(jnp.int32)
  rp = item_id.astype(jnp.int32) + USER_NUM
  rn = neg_item_id.astype(jnp.int32) + USER_NUM
  u, p, n = _gather(tables[0], tables[1], tables[2], tables[3], ru, rp, rn)
  loss, bpr, reg = _loss(u.reshape(B, 4 * D), p.reshape(B, 4 * D),
                         n.reshape(B, 4 * D))
  return (loss[0, 0], bpr[0, 0], reg[0, 0])


# speedup vs baseline: 5.5511x; 2.1769x over previous
import functools

import jax
import jax.numpy as jnp
from jax import lax
from jax.experimental import pallas as pl
from jax.experimental.pallas import tpu as pltpu
from jax.experimental.pallas import tpu_sc as plsc

USER_NUM = 25000
ITEM_NUM = 25000
N = USER_NUM + ITEM_NUM
D = 64
H = 32
L = 3
NNZ = 800000
B = 4096
DECAY = 1e-4

NC = 2
NS = 16
NP = 50048
E = 128
K = 3
NCB = 132
NITER = NCB // 2
EPT = E * K * NCB
PADN = EPT * NS
RPT = NP // NS
BPW = B // (NC * NS)


def _prop_body(ego_lo, ego_hi, cv_h, row_h, zeros_h,
               side_lo, side_hi, acc, xa, xb, ra, rb, rowsa, rowsb,
               isem, gsa, gsb, ssa, ssb):
  c = lax.axis_index("c")
  s = lax.axis_index("s")

  def half(ego_h, side_h):
    pltpu.sync_copy(zeros_h.at[pl.ds(s * RPT, RPT)],
                    acc.at[pl.ds(s * RPT, RPT)])
    plsc.subcore_barrier()

    def issue_idx(b_idx, x, r):
      pltpu.async_copy(cv_h.at[s, b_idx], x, isem)
      pltpu.async_copy(row_h.at[s, b_idx], r, isem)

    def wait_idx(b_idx, x, r):
      pltpu.make_async_copy(cv_h.at[s, b_idx], x, isem).wait()
      pltpu.make_async_copy(row_h.at[s, b_idx], r, isem).wait()

    def issue_gathers(x, rows, sem):
      for k2 in range(K):
        pltpu.async_copy(ego_h.at[x.at[2 * k2]],
                         rows.at[pl.ds(k2 * E, E)], sem)

    def drain_gathers(x, rows, sem):
      for k2 in range(K):
        pltpu.make_async_copy(ego_h.at[x.at[2 * k2]],
                              rows.at[pl.ds(k2 * E, E)], sem).wait()

    def issue_scatters(rows, r, sem):
      for k2 in range(K):
        pltpu.async_copy(rows.at[pl.ds(k2 * E, E)], acc.at[r.at[k2]],
                         sem, add=True)

    def drain_scatters(rows, r, sem):
      for k2 in range(K):
        pltpu.make_async_copy(rows.at[pl.ds(k2 * E, E)], acc.at[r.at[k2]],
                              sem).wait()

    def multiply(rows, x):
      def mbody(g, carry):
        k2 = g >> 3
        gg = g - (k2 << 3)
        vi = x[2 * k2 + 1, pl.ds(gg * 16, 16)]
        vf = plsc.bitcast(vi, jnp.float32)
        for j in range(16):
          v = vf[j]
          e = k2 * E + gg * 16 + j
          for h2 in range(H // 16):
            rows[e, pl.ds(h2 * 16, 16)] = rows[e, pl.ds(h2 * 16, 16)] * v
        return carry

      lax.fori_loop(0, K * (E // 16), mbody, 0)

    pltpu.sync_copy(cv_h.at[s, 0], xa)
    pltpu.sync_copy(row_h.at[s, 0], ra)
    issue_gathers(xa, rowsa, gsa)

    def body(i, carry):
      b_idx = 2 * i + 1
      pl.when(i > 0)(lambda: drain_scatters(rowsb, rb, ssb))
      issue_idx(b_idx, xb, rb)
      drain_gathers(xa, rowsa, gsa)
      multiply(rowsa, xa)
      issue_scatters(rowsa, ra, ssa)
      wait_idx(b_idx, xb, rb)
      issue_gathers(xb, rowsb, gsb)
      drain_scatters(rowsa, ra, ssa)

      def prefetch_a():
        issue_idx(2 * i + 2, xa, ra)

      pl.when(i < NITER - 1)(prefetch_a)
      drain_gathers(xb, rowsb, gsb)
      multiply(rowsb, xb)
      issue_scatters(rowsb, rb, ssb)

      def start_a():
        wait_idx(2 * i + 2, xa, ra)
        issue_gathers(xa, rowsa, gsa)

      pl.when(i < NITER - 1)(start_a)
      return carry

    lax.fori_loop(0, NITER, body, 0)
    drain_scatters(rowsb, rb, ssb)
    plsc.subcore_barrier()
    pltpu.sync_copy(acc.at[pl.ds(s * RPT, RPT)],
                    side_h.at[pl.ds(s * RPT, RPT)])

  pl.when(c == 0)(lambda: half(ego_lo, side_lo))
  pl.when(c == 1)(lambda: half(ego_hi, side_hi))


_prop = pl.kernel(
    _prop_body,
    out_type=[jax.ShapeDtypeStruct((NP, H), jnp.float32),
              jax.ShapeDtypeStruct((NP, H), jnp.float32)],
    mesh=plsc.VectorSubcoreMesh(core_axis_name="c", subcore_axis_name="s"),
    compiler_params=pltpu.CompilerParams(use_tc_tiling_on_sc=False,
                                         needs_layout_passes=False),
    scratch_types=[
        pltpu.VMEM_SHARED((NP, H), jnp.float32),
        pltpu.VMEM((2 * K, E), jnp.int32),
        pltpu.VMEM((2 * K, E), jnp.int32),
        pltpu.VMEM((K, E), jnp.int32),
        pltpu.VMEM((K, E), jnp.int32),
        pltpu.VMEM((K * E, H), jnp.float32),
        pltpu.VMEM((K * E, H), jnp.float32),
        pltpu.SemaphoreType.DMA,
        pltpu.SemaphoreType.DMA,
        pltpu.SemaphoreType.DMA,
        pltpu.SemaphoreType.DMA,
        pltpu.SemaphoreType.DMA,
    ],
)


def _dense_body(slo, shi, elo, ehi, wg, bg, wb, bb, nlo, nhi, norm):
  wg_t = wg[0:32, :]
  wg_b = wg[32:64, :]
  wb_t = wb[0:32, :]
  wb_b = wb[32:64, :]
  dot = functools.partial(lax.dot_general,
                          dimension_numbers=(((1,), (0,)), ((), ())),
                          preferred_element_type=jnp.float32,
                          precision=lax.Precision.HIGHEST)
  sum_e = dot(slo[...], wg_t) + dot(shi[...], wg_b) + bg[0:1, :]
  bi = dot(elo[...] * slo[...], wb_t) + dot(ehi[...] * shi[...], wb_b) + bb[0:1, :]
  t = sum_e + bi
  x = jnp.where(t >= 0, t, 0.2 * t)
  nlo[...] = x[:, 0:32]
  nhi[...] = x[:, 32:64]
  nrm = jnp.maximum(jnp.sqrt(jnp.sum(x * x, axis=1, keepdims=True)), 1e-12)
  norm[...] = x / nrm


def _dense(slo, shi, elo, ehi, wg, bg, wb, bb):
  BR = 3128
  grid = NP // BR
  row_spec = pl.BlockSpec((BR, H), lambda i: (i, 0))
  w_spec = pl.BlockSpec((D, D), lambda i: (0, 0))
  b_spec = pl.BlockSpec((8, D), lambda i: (0, 0))
  return pl.pallas_call(
      _dense_body,
      grid=(grid,),
      in_specs=[row_spec, row_spec, row_spec, row_spec,
                w_spec, b_spec, w_spec, b_spec],
      out_specs=[row_spec, row_spec, pl.BlockSpec((BR, D), lambda i: (i, 0))],
      out_shape=[jax.ShapeDtypeStruct((NP, H), jnp.float32),
                 jax.ShapeDtypeStruct((NP, H), jnp.float32),
                 jax.ShapeDtypeStruct((NP, D), jnp.float32)],
  )(slo, shi, elo, ehi, wg, bg, wb, bb)


def _gather_body(t0, t1, t2, t3, ru, rp, rn, outu, outp, outn,
                 idxv, gbuf, sem):
  c = lax.axis_index("c")
  s = lax.axis_index("s")
  w = s * NC + c
  tables = [t0, t1, t2, t3]
  for ids_h, out_h in ((ru, outu), (rp, outp), (rn, outn)):
    pltpu.sync_copy(ids_h.at[pl.ds(w * BPW, BPW)], idxv)
    for t in range(4):
      pltpu.async_copy(tables[t].at[idxv], gbuf, sem).wait()
      pltpu.sync_copy(gbuf, out_h.at[pl.ds(w * BPW, BPW), t])


_gather = pl.kernel(
    _gather_body,
    out_type=[jax.ShapeDtypeStruct((B, 4, D), jnp.float32)] * 3,
    mesh=plsc.VectorSubcoreMesh(core_axis_name="c", subcore_axis_name="s"),
    compiler_params=pltpu.CompilerParams(use_tc_tiling_on_sc=False,
                                         needs_layout_passes=False),
    scratch_types=[
        pltpu.VMEM((BPW,), jnp.int32),
        pltpu.VMEM((BPW, D), jnp.float32),
        pltpu.SemaphoreType.DMA,
    ],
)


def _loss_body(u, p, n, loss, bpr, reg):
  ps = jnp.sum(u[...] * p[...], axis=1)
  ns = jnp.sum(u[...] * n[...], axis=1)
  d = ps - ns
  ls = jnp.minimum(d, 0.0) - jnp.log1p(jnp.exp(-jnp.abs(d)))
  bpr_v = -jnp.sum(ls) / B
  u0 = u[:, 0:D]
  p0 = p[:, 0:D]
  n0 = n[:, 0:D]
  reg_v = DECAY * (jnp.sum(u0 * u0) + jnp.sum(p0 * p0) + jnp.sum(n0 * n0)) \
      / 2.0 / B
  loss[0, 0] = bpr_v + reg_v
  bpr[0, 0] = bpr_v
  reg[0, 0] = reg_v


def _loss(u, p, n):
  spec = pl.BlockSpec((B, 4 * D), lambda: (0, 0))
  return pl.pallas_call(
      _loss_body,
      in_specs=[spec, spec, spec],
      out_specs=[pl.BlockSpec(memory_space=pltpu.SMEM)] * 3,
      out_shape=[jax.ShapeDtypeStruct((1, 1), jnp.float32)] * 3,
  )(u, p, n)


def kernel(user_emb, item_emb, W_gc, b_gc, W_bi, b_bi,
           edge_val, edge_row, edge_col, user_id, item_id, neg_item_id):
  ego0 = jnp.concatenate([user_emb, item_emb,
                          jnp.zeros((NP - N, D), jnp.float32)], axis=0)
  elo = ego0[:, :H]
  ehi = ego0[:, H:]

  pad = PADN - NNZ
  pad_rows = (jnp.arange(pad, dtype=jnp.int32) * 64) % N
  col_p = jnp.concatenate([edge_col.astype(jnp.int32), pad_rows])
  row_p = jnp.concatenate([edge_row.astype(jnp.int32), pad_rows])
  val_p = jnp.concatenate([edge_val, jnp.zeros((pad,), jnp.float32)])
  val_bits = lax.bitcast_convert_type(val_p, jnp.int32)
  colr = col_p.reshape(NS, NCB, K, E)
  valr = val_bits.reshape(NS, NCB, K, E)
  cv_h = jnp.stack([colr, valr], axis=3).reshape(NS, NCB, 2 * K, E)
  rowp_h = row_p.reshape(NS, NCB, K, E)
  zeros_h = jnp.zeros((NP, H), jnp.float32)

  tables = [ego0]
  for k in range(L):
    slo, shi = _prop(elo, ehi, cv_h, rowp_h, zeros_h)
    bg = jnp.broadcast_to(b_gc[k], (8, D))
    bb = jnp.broadcast_to(b_bi[k], (8, D))
    elo, ehi, nrm = _dense(slo, shi, elo, ehi, W_gc[k], bg, W_bi[k], bb)
    tables.append(nrm)

  ru = user_id.astype(jnp.int32)
  rp = item_id.astype(jnp.int32) + USER_NUM
  rn = neg_item_id.astype(jnp.int32) + USER_NUM
  u, p, n = _gather(tables[0], tables[1], tables[2], tables[3], ru, rp, rn)
  loss, bpr, reg = _loss(u.reshape(B, 4 * D), p.reshape(B, 4 * D),
                         n.reshape(B, 4 * D))
  return (loss[0, 0], bpr[0, 0], reg[0, 0])


# speedup vs baseline: 7.7248x; 1.3916x over previous
import functools

import jax
import jax.numpy as jnp
from jax import lax
from jax.experimental import pallas as pl
from jax.experimental.pallas import tpu as pltpu
from jax.experimental.pallas import tpu_sc as plsc

USER_NUM = 25000
ITEM_NUM = 25000
N = USER_NUM + ITEM_NUM
D = 64
H = 32
L = 3
NNZ = 800000
B = 4096
DECAY = 1e-4

NC = 2
NS = 16
NP = 50048
E = 128
K = 3
NCB = 132
NITER = NCB // 2
EPT = E * K * NCB
PADN = EPT * NS
RPT = NP // NS
BPW = B // (NC * NS)


def _prop_body(ego_lo, ego_hi, cv_h, row_h, zeros_h,
               side_lo, side_hi, acc, xa, xb, ra, rb, rowsa, rowsb,
               isem, gsa, gsb, ssa, ssb):
  c = lax.axis_index("c")
  s = lax.axis_index("s")

  def half(ego_h, side_h):
    pltpu.sync_copy(zeros_h.at[pl.ds(s * RPT, RPT)],
                    acc.at[pl.ds(s * RPT, RPT)])
    plsc.subcore_barrier()

    def issue_idx(b_idx, x, r):
      pltpu.async_copy(cv_h.at[s, b_idx], x, isem)
      pltpu.async_copy(row_h.at[s, b_idx], r, isem)

    def wait_idx(b_idx, x, r):
      pltpu.make_async_copy(cv_h.at[s, b_idx], x, isem).wait()
      pltpu.make_async_copy(row_h.at[s, b_idx], r, isem).wait()

    def issue_gathers(x, rows, sem):
      for k2 in range(K):
        pltpu.async_copy(ego_h.at[x.at[2 * k2]],
                         rows.at[pl.ds(k2 * E, E)], sem)

    def drain_gathers(x, rows, sem):
      for k2 in range(K):
        pltpu.make_async_copy(ego_h.at[x.at[2 * k2]],
                              rows.at[pl.ds(k2 * E, E)], sem).wait()

    def issue_scatters(rows, r, sem):
      for k2 in range(K):
        pltpu.async_copy(rows.at[pl.ds(k2 * E, E)], acc.at[r.at[k2]],
                         sem, add=True)

    def drain_scatters(rows, r, sem):
      for k2 in range(K):
        pltpu.make_async_copy(rows.at[pl.ds(k2 * E, E)], acc.at[r.at[k2]],
                              sem).wait()

    def multiply(rows, x):
      def mbody(g, carry):
        k2 = g >> 3
        gg = g - (k2 << 3)
        vi = x[2 * k2 + 1, pl.ds(gg * 16, 16)]
        vf = plsc.bitcast(vi, jnp.float32)
        for j in range(16):
          v = vf[j]
          e = k2 * E + gg * 16 + j
          for h2 in range(H // 16):
            rows[e, pl.ds(h2 * 16, 16)] = rows[e, pl.ds(h2 * 16, 16)] * v
        return carry

      lax.fori_loop(0, K * (E // 16), mbody, 0)

    pltpu.sync_copy(cv_h.at[s, 0], xa)
    pltpu.sync_copy(row_h.at[s, 0], ra)
    issue_gathers(xa, rowsa, gsa)

    def body(i, carry):
      b_idx = 2 * i + 1
      pl.when(i > 0)(lambda: drain_scatters(rowsb, rb, ssb))
      issue_idx(b_idx, xb, rb)
      drain_gathers(xa, rowsa, gsa)
      multiply(rowsa, xa)
      issue_scatters(rowsa, ra, ssa)
      wait_idx(b_idx, xb, rb)
      issue_gathers(xb, rowsb, gsb)
      drain_scatters(rowsa, ra, ssa)

      def prefetch_a():
        issue_idx(2 * i + 2, xa, ra)

      pl.when(i < NITER - 1)(prefetch_a)
      drain_gathers(xb, rowsb, gsb)
      multiply(rowsb, xb)
      issue_scatters(rowsb, rb, ssb)

      def start_a():
        wait_idx(2 * i + 2, xa, ra)
        issue_gathers(xa, rowsa, gsa)

      pl.when(i < NITER - 1)(start_a)
      return carry

    lax.fori_loop(0, NITER, body, 0)
    drain_scatters(rowsb, rb, ssb)
    plsc.subcore_barrier()
    pltpu.sync_copy(acc.at[pl.ds(s * RPT, RPT)],
                    side_h.at[pl.ds(s * RPT, RPT)])

  pl.when(c == 0)(lambda: half(ego_lo, side_lo))
  pl.when(c == 1)(lambda: half(ego_hi, side_hi))


_prop = pl.kernel(
    _prop_body,
    out_type=[jax.ShapeDtypeStruct((NP, H), jnp.float32),
              jax.ShapeDtypeStruct((NP, H), jnp.float32)],
    mesh=plsc.VectorSubcoreMesh(core_axis_name="c", subcore_axis_name="s"),
    compiler_params=pltpu.CompilerParams(use_tc_tiling_on_sc=False,
                                         needs_layout_passes=False),
    scratch_types=[
        pltpu.VMEM_SHARED((NP, H), jnp.float32),
        pltpu.VMEM((2 * K, E), jnp.int32),
        pltpu.VMEM((2 * K, E), jnp.int32),
        pltpu.VMEM((K, E), jnp.int32),
        pltpu.VMEM((K, E), jnp.int32),
        pltpu.VMEM((K * E, H), jnp.float32),
        pltpu.VMEM((K * E, H), jnp.float32),
        pltpu.SemaphoreType.DMA,
        pltpu.SemaphoreType.DMA,
        pltpu.SemaphoreType.DMA,
        pltpu.SemaphoreType.DMA,
        pltpu.SemaphoreType.DMA,
    ],
)


def _dense_body(slo, shi, elo, ehi, w, b, nlo, nhi, norm):
  side = jnp.concatenate([slo[...], shi[...]], axis=1)
  eg = jnp.concatenate([elo[...], ehi[...]], axis=1)
  xin = jnp.concatenate([side, eg * side], axis=1)
  t = lax.dot_general(xin, w[...],
                      dimension_numbers=(((1,), (0,)), ((), ())),
                      preferred_element_type=jnp.float32) + b[0:1, :]
  x = jnp.where(t >= 0, t, 0.2 * t)
  nlo[...] = x[:, 0:32]
  nhi[...] = x[:, 32:64]
  nrm = jnp.maximum(jnp.sqrt(jnp.sum(x * x, axis=1, keepdims=True)), 1e-12)
  norm[...] = x / nrm


def _dense(slo, shi, elo, ehi, w, b):
  BR = 3128
  grid = NP // BR
  row_spec = pl.BlockSpec((BR, H), lambda i: (i, 0))
  w_spec = pl.BlockSpec((2 * D, D), lambda i: (0, 0))
  b_spec = pl.BlockSpec((8, D), lambda i: (0, 0))
  return pl.pallas_call(
      _dense_body,
      grid=(grid,),
      in_specs=[row_spec, row_spec, row_spec, row_spec, w_spec, b_spec],
      out_specs=[row_spec, row_spec, pl.BlockSpec((BR, D), lambda i: (i, 0))],
      out_shape=[jax.ShapeDtypeStruct((NP, H), jnp.float32),
                 jax.ShapeDtypeStruct((NP, H), jnp.float32),
                 jax.ShapeDtypeStruct((NP, D), jnp.float32)],
  )(slo, shi, elo, ehi, w, b)


def _gather_body(t0, t1, t2, t3, ru, rp, rn, outu, outp, outn,
                 idxv, gbuf, sem):
  c = lax.axis_index("c")
  s = lax.axis_index("s")
  w = s * NC + c
  tables = [t0, t1, t2, t3]
  for ids_h, out_h in ((ru, outu), (rp, outp), (rn, outn)):
    pltpu.sync_copy(ids_h.at[pl.ds(w * BPW, BPW)], idxv)
    for t in range(4):
      pltpu.async_copy(tables[t].at[idxv], gbuf, sem).wait()
      pltpu.sync_copy(gbuf, out_h.at[pl.ds(w * BPW, BPW), t])


_gather = pl.kernel(
    _gather_body,
    out_type=[jax.ShapeDtypeStruct((B, 4, D), jnp.float32)] * 3,
    mesh=plsc.VectorSubcoreMesh(core_axis_name="c", subcore_axis_name="s"),
    compiler_params=pltpu.CompilerParams(use_tc_tiling_on_sc=False,
                                         needs_layout_passes=False),
    scratch_types=[
        pltpu.VMEM((BPW,), jnp.int32),
        pltpu.VMEM((BPW, D), jnp.float32),
        pltpu.SemaphoreType.DMA,
    ],
)


def _loss_body(u, p, n, loss, bpr, reg):
  ps = jnp.sum(u[...] * p[...], axis=1)
  ns = jnp.sum(u[...] * n[...], axis=1)
  d = ps - ns
  ls = jnp.minimum(d, 0.0) - jnp.log1p(jnp.exp(-jnp.abs(d)))
  bpr_v = -jnp.sum(ls) / B
  u0 = u[:, 0:D]
  p0 = p[:, 0:D]
  n0 = n[:, 0:D]
  reg_v = DECAY * (jnp.sum(u0 * u0) + jnp.sum(p0 * p0) + jnp.sum(n0 * n0)) \
      / 2.0 / B
  loss[0, 0] = bpr_v + reg_v
  bpr[0, 0] = bpr_v
  reg[0, 0] = reg_v


def _loss(u, p, n):
  spec = pl.BlockSpec((B, 4 * D), lambda: (0, 0))
  return pl.pallas_call(
      _loss_body,
      in_specs=[spec, spec, spec],
      out_specs=[pl.BlockSpec(memory_space=pltpu.SMEM)] * 3,
      out_shape=[jax.ShapeDtypeStruct((1, 1), jnp.float32)] * 3,
  )(u, p, n)


def kernel(user_emb, item_emb, W_gc, b_gc, W_bi, b_bi,
           edge_val, edge_row, edge_col, user_id, item_id, neg_item_id):
  ego0 = jnp.concatenate([user_emb, item_emb,
                          jnp.zeros((NP - N, D), jnp.float32)], axis=0)
  elo = ego0[:, :H]
  ehi = ego0[:, H:]

  pad = PADN - NNZ
  pad_rows = (jnp.arange(pad, dtype=jnp.int32) * 64) % N
  col_p = jnp.concatenate([edge_col.astype(jnp.int32), pad_rows])
  row_p = jnp.concatenate([edge_row.astype(jnp.int32), pad_rows])
  val_p = jnp.concatenate([edge_val, jnp.zeros((pad,), jnp.float32)])
  val_bits = lax.bitcast_convert_type(val_p, jnp.int32)
  colr = col_p.reshape(NS, NCB, K, E)
  valr = val_bits.reshape(NS, NCB, K, E)
  cv_h = jnp.stack([colr, valr], axis=3).reshape(NS, NCB, 2 * K, E)
  rowp_h = row_p.reshape(NS, NCB, K, E)
  zeros_h = jnp.zeros((NP, H), jnp.float32)

  tables = [ego0]
  for k in range(L):
    slo, shi = _prop(elo, ehi, cv_h, rowp_h, zeros_h)
    w = jnp.concatenate([W_gc[k], W_bi[k]], axis=0)
    b = jnp.broadcast_to(b_gc[k] + b_bi[k], (8, D))
    elo, ehi, nrm = _dense(slo, shi, elo, ehi, w, b)
    tables.append(nrm)

  ru = user_id.astype(jnp.int32)
  rp = item_id.astype(jnp.int32) + USER_NUM
  rn = neg_item_id.astype(jnp.int32) + USER_NUM
  u, p, n = _gather(tables[0], tables[1], tables[2], tables[3], ru, rp, rn)
  loss, bpr, reg = _loss(u.reshape(B, 4 * D), p.reshape(B, 4 * D),
                         n.reshape(B, 4 * D))
  return (loss[0, 0], bpr[0, 0], reg[0, 0])


# speedup vs baseline: 8.1961x; 1.0610x over previous
import functools

import jax
import jax.numpy as jnp
from jax import lax
from jax.experimental import pallas as pl
from jax.experimental.pallas import tpu as pltpu
from jax.experimental.pallas import tpu_sc as plsc

USER_NUM = 25000
ITEM_NUM = 25000
N = USER_NUM + ITEM_NUM
D = 64
H = 32
L = 3
NNZ = 800000
B = 4096
DECAY = 1e-4

NC = 2
NS = 16
NP = 50048
E = 128
K = 3
NCB = 132
NITER = NCB // 2
EPT = E * K * NCB
PADN = EPT * NS
RPT = NP // NS
BPW = B // (NC * NS)


def _prop_body(ego_lo, ego_hi, col_h, val_h, row_h,
               side_lo, side_hi, acc, ca, cb, va, vb, ra, rb, rowsa, rowsb,
               isem, gsa, gsb, ssa, ssb):
  c = lax.axis_index("c")
  s = lax.axis_index("s")
  KE = K * E

  def half(ego_h, side_h):
    zero = jnp.zeros((16,), jnp.float32)

    def zbody(r0, carry):
      for h2 in range(H // 16):
        rowsa[r0, pl.ds(h2 * 16, 16)] = zero
      return carry

    lax.fori_loop(0, KE, zbody, 0)
    for i in range(RPT // KE):
      pltpu.sync_copy(rowsa, acc.at[pl.ds(s * RPT + i * KE, KE)])
    rem = RPT - (RPT // KE) * KE
    if rem:
      pltpu.sync_copy(rowsa.at[pl.ds(0, rem)],
                      acc.at[pl.ds(s * RPT + (RPT // KE) * KE, rem)])
    plsc.subcore_barrier()

    def issue_idx(b_idx, cv, vv, r):
      base = (s * NCB + b_idx) * KE
      pltpu.async_copy(col_h.at[pl.ds(base, KE)], cv, isem)
      pltpu.async_copy(val_h.at[pl.ds(base, KE)], vv, isem)
      pltpu.async_copy(row_h.at[s, b_idx], r, isem)

    def wait_idx(b_idx, cv, vv, r):
      base = (s * NCB + b_idx) * KE
      pltpu.make_async_copy(col_h.at[pl.ds(base, KE)], cv, isem).wait()
      pltpu.make_async_copy(val_h.at[pl.ds(base, KE)], vv, isem).wait()
      pltpu.make_async_copy(row_h.at[s, b_idx], r, isem).wait()

    def issue_gathers(cv, rows, sem):
      for k2 in range(K):
        pltpu.async_copy(ego_h.at[cv.at[pl.ds(k2 * E, E)]],
                         rows.at[pl.ds(k2 * E, E)], sem)

    def drain_gathers(cv, rows, sem):
      for k2 in range(K):
        pltpu.make_async_copy(ego_h.at[cv.at[pl.ds(k2 * E, E)]],
                              rows.at[pl.ds(k2 * E, E)], sem).wait()

    def issue_scatters(rows, r, sem):
      for k2 in range(K):
        pltpu.async_copy(rows.at[pl.ds(k2 * E, E)], acc.at[r.at[k2]],
                         sem, add=True)

    def drain_scatters(rows, r, sem):
      for k2 in range(K):
        pltpu.make_async_copy(rows.at[pl.ds(k2 * E, E)], acc.at[r.at[k2]],
                              sem).wait()

    def multiply(rows, vv):
      def mbody(g, carry):
        vf = vv[pl.ds(g * 16, 16)]
        for j in range(16):
          v = vf[j]
          e = g * 16 + j
          for h2 in range(H // 16):
            rows[e, pl.ds(h2 * 16, 16)] = rows[e, pl.ds(h2 * 16, 16)] * v
        return carry

      lax.fori_loop(0, KE // 16, mbody, 0)

    base0 = s * NCB * KE
    pltpu.sync_copy(col_h.at[pl.ds(base0, KE)], ca)
    pltpu.sync_copy(val_h.at[pl.ds(base0, KE)], va)
    pltpu.sync_copy(row_h.at[s, 0], ra)
    issue_gathers(ca, rowsa, gsa)

    def body(i, carry):
      b_idx = 2 * i + 1
      pl.when(i > 0)(lambda: drain_scatters(rowsb, rb, ssb))
      issue_idx(b_idx, cb, vb, rb)
      drain_gathers(ca, rowsa, gsa)
      multiply(rowsa, va)
      issue_scatters(rowsa, ra, ssa)
      wait_idx(b_idx, cb, vb, rb)
      issue_gathers(cb, rowsb, gsb)
      drain_scatters(rowsa, ra, ssa)

      def prefetch_a():
        issue_idx(2 * i + 2, ca, va, ra)

      pl.when(i < NITER - 1)(prefetch_a)
      drain_gathers(cb, rowsb, gsb)
      multiply(rowsb, vb)
      issue_scatters(rowsb, rb, ssb)

      def start_a():
        wait_idx(2 * i + 2, ca, va, ra)
        issue_gathers(ca, rowsa, gsa)

      pl.when(i < NITER - 1)(start_a)
      return carry

    lax.fori_loop(0, NITER, body, 0)
    drain_scatters(rowsb, rb, ssb)
    plsc.subcore_barrier()
    pltpu.sync_copy(acc.at[pl.ds(s * RPT, RPT)],
                    side_h.at[pl.ds(s * RPT, RPT)])

  pl.when(c == 0)(lambda: half(ego_lo, side_lo))
  pl.when(c == 1)(lambda: half(ego_hi, side_hi))


_prop = pl.kernel(
    _prop_body,
    out_type=[jax.ShapeDtypeStruct((NP, H), jnp.float32),
              jax.ShapeDtypeStruct((NP, H), jnp.float32)],
    mesh=plsc.VectorSubcoreMesh(core_axis_name="c", subcore_axis_name="s"),
    compiler_params=pltpu.CompilerParams(use_tc_tiling_on_sc=False,
                                         needs_layout_passes=False),
    scratch_types=[
        pltpu.VMEM_SHARED((NP, H), jnp.float32),
        pltpu.VMEM((K * E,), jnp.int32),
        pltpu.VMEM((K * E,), jnp.int32),
        pltpu.VMEM((K * E,), jnp.float32),
        pltpu.VMEM((K * E,), jnp.float32),
        pltpu.VMEM((K, E), jnp.int32),
        pltpu.VMEM((K, E), jnp.int32),
        pltpu.VMEM((K * E, H), jnp.float32),
        pltpu.VMEM((K * E, H), jnp.float32),
        pltpu.SemaphoreType.DMA,
        pltpu.SemaphoreType.DMA,
        pltpu.SemaphoreType.DMA,
        pltpu.SemaphoreType.DMA,
        pltpu.SemaphoreType.DMA,
    ],
)


def _dense_body(slo, shi, elo, ehi, w, b, nlo, nhi, norm):
  side = jnp.concatenate([slo[...], shi[...]], axis=1)
  eg = jnp.concatenate([elo[...], ehi[...]], axis=1)
  xin = jnp.concatenate([side, eg * side], axis=1)
  t = lax.dot_general(xin, w[...],
                      dimension_numbers=(((1,), (0,)), ((), ())),
                      preferred_element_type=jnp.float32) + b[0:1, :]
  x = jnp.where(t >= 0, t, 0.2 * t)
  nlo[...] = x[:, 0:32]
  nhi[...] = x[:, 32:64]
  nrm = jnp.maximum(jnp.sqrt(jnp.sum(x * x, axis=1, keepdims=True)), 1e-12)
  norm[...] = x / nrm


def _dense(slo, shi, elo, ehi, w, b):
  BR = 3128
  grid = NP // BR
  row_spec = pl.BlockSpec((BR, H), lambda i: (i, 0))
  w_spec = pl.BlockSpec((2 * D, D), lambda i: (0, 0))
  b_spec = pl.BlockSpec((8, D), lambda i: (0, 0))
  return pl.pallas_call(
      _dense_body,
      grid=(grid,),
      in_specs=[row_spec, row_spec, row_spec, row_spec, w_spec, b_spec],
      out_specs=[row_spec, row_spec, pl.BlockSpec((BR, D), lambda i: (i, 0))],
      out_shape=[jax.ShapeDtypeStruct((NP, H), jnp.float32),
                 jax.ShapeDtypeStruct((NP, H), jnp.float32),
                 jax.ShapeDtypeStruct((NP, D), jnp.float32)],
  )(slo, shi, elo, ehi, w, b)


def _gather_body(t0, t1, t2, t3, ru, rp, rn, outu, outp, outn,
                 idxv, gbuf, sem):
  c = lax.axis_index("c")
  s = lax.axis_index("s")
  w = s * NC + c
  tables = [t0, t1, t2, t3]
  for ids_h, out_h in ((ru, outu), (rp, outp), (rn, outn)):
    pltpu.sync_copy(ids_h.at[pl.ds(w * BPW, BPW)], idxv)
    for t in range(4):
      pltpu.async_copy(tables[t].at[idxv], gbuf, sem).wait()
      pltpu.sync_copy(gbuf, out_h.at[pl.ds(w * BPW, BPW), pl.ds(t * D, D)])


_gather = pl.kernel(
    _gather_body,
    out_type=[jax.ShapeDtypeStruct((B, 4 * D), jnp.float32)] * 3,
    mesh=plsc.VectorSubcoreMesh(core_axis_name="c", subcore_axis_name="s"),
    compiler_params=pltpu.CompilerParams(use_tc_tiling_on_sc=False,
                                         needs_layout_passes=False),
    scratch_types=[
        pltpu.VMEM((BPW,), jnp.int32),
        pltpu.VMEM((BPW, D), jnp.float32),
        pltpu.SemaphoreType.DMA,
    ],
)


def _loss_body(u, p, n, loss, bpr, reg):
  ps = jnp.sum(u[...] * p[...], axis=1)
  ns = jnp.sum(u[...] * n[...], axis=1)
  d = ps - ns
  ls = jnp.minimum(d, 0.0) - jnp.log1p(jnp.exp(-jnp.abs(d)))
  bpr_v = -jnp.sum(ls) / B
  u0 = u[:, 0:D]
  p0 = p[:, 0:D]
  n0 = n[:, 0:D]
  reg_v = DECAY * (jnp.sum(u0 * u0) + jnp.sum(p0 * p0) + jnp.sum(n0 * n0)) \
      / 2.0 / B
  loss[0, 0] = bpr_v + reg_v
  bpr[0, 0] = bpr_v
  reg[0, 0] = reg_v


def _loss(u, p, n):
  spec = pl.BlockSpec((B, 4 * D), lambda: (0, 0))
  return pl.pallas_call(
      _loss_body,
      in_specs=[spec, spec, spec],
      out_specs=[pl.BlockSpec(memory_space=pltpu.SMEM)] * 3,
      out_shape=[jax.ShapeDtypeStruct((1, 1), jnp.float32)] * 3,
  )(u, p, n)


def kernel(user_emb, item_emb, W_gc, b_gc, W_bi, b_bi,
           edge_val, edge_row, edge_col, user_id, item_id, neg_item_id):
  ego0 = jnp.concatenate([user_emb, item_emb,
                          jnp.zeros((NP - N, D), jnp.float32)], axis=0)
  elo = ego0[:, :H]
  ehi = ego0[:, H:]

  pad = PADN - NNZ
  pad_rows = (jnp.arange(pad, dtype=jnp.int32) * 64) % N
  col_p = jnp.concatenate([edge_col.astype(jnp.int32), pad_rows])
  row_p = jnp.concatenate([edge_row.astype(jnp.int32), pad_rows])
  val_p = jnp.concatenate([edge_val, jnp.zeros((pad,), jnp.float32)])
  rowp_h = row_p.reshape(NS, NCB, K, E)

  tables = [ego0]
  for k in range(L):
    slo, shi = _prop(elo, ehi, col_p, val_p, rowp_h)
    w = jnp.concatenate([W_gc[k], W_bi[k]], axis=0)
    b = jnp.broadcast_to(b_gc[k] + b_bi[k], (8, D))
    elo, ehi, nrm = _dense(slo, shi, elo, ehi, w, b)
    tables.append(nrm)

  ru = user_id.astype(jnp.int32)
  rp = item_id.astype(jnp.int32) + USER_NUM
  rn = neg_item_id.astype(jnp.int32) + USER_NUM
  u, p, n = _gather(tables[0], tables[1], tables[2], tables[3], ru, rp, rn)
  loss, bpr, reg = _loss(u, p, n)
  return (loss[0, 0], bpr[0, 0], reg[0, 0])


# speedup vs baseline: 8.3725x; 1.0215x over previous
import functools

import jax
import jax.numpy as jnp
from jax import lax
from jax.experimental import pallas as pl
from jax.experimental.pallas import tpu as pltpu
from jax.experimental.pallas import tpu_sc as plsc

USER_NUM = 25000
ITEM_NUM = 25000
N = USER_NUM + ITEM_NUM
D = 64
H = 32
L = 3
NNZ = 800000
B = 4096
DECAY = 1e-4

NC = 2
NS = 16
NP = 50048
E = 128
K = 3
NCB = 132
NITER = NCB // 2
EPT = E * K * NCB
PADN = EPT * NS
RPT = NP // NS
BPW = B // (NC * NS)


def _prop_body(ego_lo, ego_hi, col_h, val_h, row_h,
               side_lo, side_hi, acc, ca, cb, va, vb, ra, rb, rowsa, rowsb,
               isem, gsa, gsb, ssa, ssb):
  c = lax.axis_index("c")
  s = lax.axis_index("s")
  KE = K * E

  def half(ego_h, side_h):
    zero = jnp.zeros((16,), jnp.float32)

    def zbody(r0, carry):
      for h2 in range(H // 16):
        rowsa[r0, pl.ds(h2 * 16, 16)] = zero
      return carry

    lax.fori_loop(0, KE, zbody, 0)
    for i in range(RPT // KE):
      pltpu.sync_copy(rowsa, acc.at[pl.ds(s * RPT + i * KE, KE)])
    rem = RPT - (RPT // KE) * KE
    if rem:
      pltpu.sync_copy(rowsa.at[pl.ds(0, rem)],
                      acc.at[pl.ds(s * RPT + (RPT // KE) * KE, rem)])
    plsc.subcore_barrier()

    def issue_idx(b_idx, cv, vv, r):
      base = (s * NCB + b_idx) * KE
      pltpu.async_copy(col_h.at[pl.ds(base, KE)], cv, isem)
      pltpu.async_copy(val_h.at[pl.ds(base, KE)], vv, isem)
      pltpu.async_copy(row_h.at[s, b_idx], r, isem)

    def wait_idx(b_idx, cv, vv, r):
      base = (s * NCB + b_idx) * KE
      pltpu.make_async_copy(col_h.at[pl.ds(base, KE)], cv, isem).wait()
      pltpu.make_async_copy(val_h.at[pl.ds(base, KE)], vv, isem).wait()
      pltpu.make_async_copy(row_h.at[s, b_idx], r, isem).wait()

    def issue_gathers(cv, rows, sem):
      for k2 in range(K):
        pltpu.async_copy(ego_h.at[cv.at[pl.ds(k2 * E, E)]],
                         rows.at[pl.ds(k2 * E, E)], sem)

    def drain_gathers(cv, rows, sem):
      for k2 in range(K):
        pltpu.make_async_copy(ego_h.at[cv.at[pl.ds(k2 * E, E)]],
                              rows.at[pl.ds(k2 * E, E)], sem).wait()

    def issue_scatters(rows, r, sem):
      for k2 in range(K):
        pltpu.async_copy(rows.at[pl.ds(k2 * E, E)], acc.at[r.at[k2]],
                         sem, add=True)

    def drain_scatters(rows, r, sem):
      for k2 in range(K):
        pltpu.make_async_copy(rows.at[pl.ds(k2 * E, E)], acc.at[r.at[k2]],
                              sem).wait()

    def multiply(rows, vv):
      def mbody(g, carry):
        vf = vv[pl.ds(g * 16, 16)]
        for j in range(16):
          v = vf[j]
          e = g * 16 + j
          for h2 in range(H // 16):
            rows[e, pl.ds(h2 * 16, 16)] = rows[e, pl.ds(h2 * 16, 16)] * v
        return carry

      lax.fori_loop(0, KE // 16, mbody, 0)

    base0 = s * NCB * KE
    pltpu.sync_copy(col_h.at[pl.ds(base0, KE)], ca)
    pltpu.sync_copy(val_h.at[pl.ds(base0, KE)], va)
    pltpu.sync_copy(row_h.at[s, 0], ra)
    issue_gathers(ca, rowsa, gsa)

    def body(i, carry):
      b_idx = 2 * i + 1
      pl.when(i > 0)(lambda: drain_scatters(rowsb, rb, ssb))
      issue_idx(b_idx, cb, vb, rb)
      drain_gathers(ca, rowsa, gsa)
      multiply(rowsa, va)
      issue_scatters(rowsa, ra, ssa)
      wait_idx(b_idx, cb, vb, rb)
      issue_gathers(cb, rowsb, gsb)
      drain_scatters(rowsa, ra, ssa)

      def prefetch_a():
        issue_idx(2 * i + 2, ca, va, ra)

      pl.when(i < NITER - 1)(prefetch_a)
      drain_gathers(cb, rowsb, gsb)
      multiply(rowsb, vb)
      issue_scatters(rowsb, rb, ssb)

      def start_a():
        wait_idx(2 * i + 2, ca, va, ra)
        issue_gathers(ca, rowsa, gsa)

      pl.when(i < NITER - 1)(start_a)
      return carry

    lax.fori_loop(0, NITER, body, 0)
    drain_scatters(rowsb, rb, ssb)
    plsc.subcore_barrier()
    pltpu.sync_copy(acc.at[pl.ds(s * RPT, RPT)],
                    side_h.at[pl.ds(s * RPT, RPT)])

  pl.when(c == 0)(lambda: half(ego_lo, side_lo))
  pl.when(c == 1)(lambda: half(ego_hi, side_hi))


_prop = pl.kernel(
    _prop_body,
    out_type=[jax.ShapeDtypeStruct((NP, H), jnp.float32),
              jax.ShapeDtypeStruct((NP, H), jnp.float32)],
    mesh=plsc.VectorSubcoreMesh(core_axis_name="c", subcore_axis_name="s"),
    compiler_params=pltpu.CompilerParams(use_tc_tiling_on_sc=False,
                                         needs_layout_passes=False),
    scratch_types=[
        pltpu.VMEM_SHARED((NP, H), jnp.float32),
        pltpu.VMEM((K * E,), jnp.int32),
        pltpu.VMEM((K * E,), jnp.int32),
        pltpu.VMEM((K * E,), jnp.float32),
        pltpu.VMEM((K * E,), jnp.float32),
        pltpu.VMEM((K, E), jnp.int32),
        pltpu.VMEM((K, E), jnp.int32),
        pltpu.VMEM((K * E, H), jnp.float32),
        pltpu.VMEM((K * E, H), jnp.float32),
        pltpu.SemaphoreType.DMA,
        pltpu.SemaphoreType.DMA,
        pltpu.SemaphoreType.DMA,
        pltpu.SemaphoreType.DMA,
        pltpu.SemaphoreType.DMA,
    ],
)


def _dense_core(slo, shi, elo, ehi, w, b):
  side = jnp.concatenate([slo[...], shi[...]], axis=1)
  eg = jnp.concatenate([elo[...], ehi[...]], axis=1)
  xin = jnp.concatenate([side, eg * side], axis=1)
  t = lax.dot_general(xin, w[...],
                      dimension_numbers=(((1,), (0,)), ((), ())),
                      preferred_element_type=jnp.float32) + b[0:1, :]
  x = jnp.where(t >= 0, t, 0.2 * t)
  nrm = jnp.maximum(jnp.sqrt(jnp.sum(x * x, axis=1, keepdims=True)), 1e-12)
  return x, jnp.concatenate([x / nrm, jnp.zeros_like(x)], axis=1)


def _dense_body(slo, shi, elo, ehi, w, b, nlo, nhi, norm):
  x, n128 = _dense_core(slo, shi, elo, ehi, w, b)
  nlo[...] = x[:, 0:32]
  nhi[...] = x[:, 32:64]
  norm[...] = n128


def _dense_last_body(slo, shi, elo, ehi, w, b, norm):
  _, n128 = _dense_core(slo, shi, elo, ehi, w, b)
  norm[...] = n128


BR = 3128
_row_spec = pl.BlockSpec((BR, H), lambda i: (i, 0))
_w_spec = pl.BlockSpec((2 * D, D), lambda i: (0, 0))
_b_spec = pl.BlockSpec((8, D), lambda i: (0, 0))
_n_spec = pl.BlockSpec((BR, 2 * D), lambda i: (i, 0))


def _dense(slo, shi, elo, ehi, w, b):
  return pl.pallas_call(
      _dense_body,
      grid=(NP // BR,),
      in_specs=[_row_spec, _row_spec, _row_spec, _row_spec, _w_spec, _b_spec],
      out_specs=[_row_spec, _row_spec, _n_spec],
      out_shape=[jax.ShapeDtypeStruct((NP, H), jnp.float32),
                 jax.ShapeDtypeStruct((NP, H), jnp.float32),
                 jax.ShapeDtypeStruct((NP, 2 * D), jnp.float32)],
  )(slo, shi, elo, ehi, w, b)


def _dense_last(slo, shi, elo, ehi, w, b):
  return pl.pallas_call(
      _dense_last_body,
      grid=(NP // BR,),
      in_specs=[_row_spec, _row_spec, _row_spec, _row_spec, _w_spec, _b_spec],
      out_specs=[_n_spec],
      out_shape=[jax.ShapeDtypeStruct((NP, 2 * D), jnp.float32)],
  )(slo, shi, elo, ehi, w, b)


def _gather_body(t0, t1, t2, t3, ru, rp, rn, outu, outp, outn,
                 idxv, gbuf, sem):
  c = lax.axis_index("c")
  s = lax.axis_index("s")
  w = s * NC + c
  tables = [t0, t1, t2, t3]
  for ids_h, out_h in ((ru, outu), (rp, outp), (rn, outn)):
    pltpu.sync_copy(ids_h.at[pl.ds(w * BPW, BPW)], idxv)
    for t in range(4):
      pltpu.async_copy(tables[t].at[idxv], gbuf, sem).wait()
      pltpu.sync_copy(gbuf, out_h.at[t, pl.ds(w * BPW, BPW)])


_gather = pl.kernel(
    _gather_body,
    out_type=[jax.ShapeDtypeStruct((4, B, 2 * D), jnp.float32)] * 3,
    mesh=plsc.VectorSubcoreMesh(core_axis_name="c", subcore_axis_name="s"),
    compiler_params=pltpu.CompilerParams(needs_layout_passes=False),
    scratch_types=[
        pltpu.VMEM((BPW,), jnp.int32),
        pltpu.VMEM((BPW, 2 * D), jnp.float32),
        pltpu.SemaphoreType.DMA,
    ],
)


def _loss_body(u, p, n, loss, bpr, reg):
  ps = 0.0
  ns = 0.0
  for t in range(4):
    ps = ps + jnp.sum(u[t, :, 0:D] * p[t, :, 0:D], axis=1)
    ns = ns + jnp.sum(u[t, :, 0:D] * n[t, :, 0:D], axis=1)
  d = ps - ns
  ls = jnp.minimum(d, 0.0) - jnp.log1p(jnp.exp(-jnp.abs(d)))
  bpr_v = -jnp.sum(ls) / B
  u0 = u[0, :, 0:D]
  p0 = p[0, :, 0:D]
  n0 = n[0, :, 0:D]
  reg_v = DECAY * (jnp.sum(u0 * u0) + jnp.sum(p0 * p0) + jnp.sum(n0 * n0)) \
      / 2.0 / B
  loss[0, 0] = bpr_v + reg_v
  bpr[0, 0] = bpr_v
  reg[0, 0] = reg_v


def _loss(u, p, n):
  spec = pl.BlockSpec((4, B, 2 * D), lambda: (0, 0, 0))
  return pl.pallas_call(
      _loss_body,
      in_specs=[spec, spec, spec],
      out_specs=[pl.BlockSpec(memory_space=pltpu.SMEM)] * 3,
      out_shape=[jax.ShapeDtypeStruct((1, 1), jnp.float32)] * 3,
  )(u, p, n)


def kernel(user_emb, item_emb, W_gc, b_gc, W_bi, b_bi,
           edge_val, edge_row, edge_col, user_id, item_id, neg_item_id):
  ego0 = jnp.concatenate([user_emb, item_emb,
                          jnp.zeros((NP - N, D), jnp.float32)], axis=0)
  elo = ego0[:, :H]
  ehi = ego0[:, H:]

  pad = PADN - NNZ
  pad_rows = (jnp.arange(pad, dtype=jnp.int32) * 64) % N
  col_p = jnp.concatenate([edge_col.astype(jnp.int32), pad_rows])
  row_p = jnp.concatenate([edge_row.astype(jnp.int32), pad_rows])
  val_p = jnp.concatenate([edge_val, jnp.zeros((pad,), jnp.float32)])
  rowp_h = row_p.reshape(NS, NCB, K, E)

  tables = [jnp.concatenate([ego0, jnp.zeros((NP, D), jnp.float32)], axis=1)]
  for k in range(L):
    slo, shi = _prop(elo, ehi, col_p, val_p, rowp_h)
    w = jnp.concatenate([W_gc[k], W_bi[k]], axis=0)
    b = jnp.broadcast_to(b_gc[k] + b_bi[k], (8, D))
    if k < L - 1:
      elo, ehi, nrm = _dense(slo, shi, elo, ehi, w, b)
    else:
      (nrm,) = _dense_last(slo, shi, elo, ehi, w, b)
    tables.append(nrm)

  ru = user_id.astype(jnp.int32)
  rp = item_id.astype(jnp.int32) + USER_NUM
  rn = neg_item_id.astype(jnp.int32) + USER_NUM
  u, p, n = _gather(tables[0], tables[1], tables[2], tables[3], ru, rp, rn)
  loss, bpr, reg = _loss(u, p, n)
  return (loss[0, 0], bpr[0, 0], reg[0, 0])


# speedup vs baseline: 9.1067x; 1.0877x over previous
import functools

import jax
import jax.numpy as jnp
from jax import lax
from jax.experimental import pallas as pl
from jax.experimental.pallas import tpu as pltpu
from jax.experimental.pallas import tpu_sc as plsc

USER_NUM = 25000
ITEM_NUM = 25000
N = USER_NUM + ITEM_NUM
D = 64
H = 32
L = 3
NNZ = 800000
B = 4096
DECAY = 1e-4

NC = 2
NS = 16
NP = 50048
E = 128
K = 3
NCB = 132
NITER = NCB // 2
EPT = E * K * NCB
PADN = EPT * NS
RPT = NP // NS
BPW = B // (NC * NS)


def _prop_body(ego_lo, ego_hi, col_h, val_h, row_h,
               side_lo, side_hi, acc, ca, cb, va, vb, ra, rb, rowsa, rowsb,
               isem, rsa, rsb, gsa, gsb, ssa, ssb):
  c = lax.axis_index("c")
  s = lax.axis_index("s")
  KE = K * E

  def half(ego_h, side_h):
    zero = jnp.zeros((16,), jnp.float32)

    def zbody(r0, carry):
      for h2 in range(H // 16):
        rowsa[r0, pl.ds(h2 * 16, 16)] = zero
      return carry

    lax.fori_loop(0, KE, zbody, 0)
    for i in range(RPT // KE):
      pltpu.sync_copy(rowsa, acc.at[pl.ds(s * RPT + i * KE, KE)])
    rem = RPT - (RPT // KE) * KE
    if rem:
      pltpu.sync_copy(rowsa.at[pl.ds(0, rem)],
                      acc.at[pl.ds(s * RPT + (RPT // KE) * KE, rem)])
    plsc.subcore_barrier()

    def issue_cv(b_idx, cv, vv):
      base = (s * NCB + b_idx) * KE
      pltpu.async_copy(col_h.at[pl.ds(base, KE)], cv, isem)
      pltpu.async_copy(val_h.at[pl.ds(base, KE)], vv, isem)

    def wait_cv(b_idx, cv, vv):
      base = (s * NCB + b_idx) * KE
      pltpu.make_async_copy(col_h.at[pl.ds(base, KE)], cv, isem).wait()
      pltpu.make_async_copy(val_h.at[pl.ds(base, KE)], vv, isem).wait()

    def issue_row(b_idx, r, rsem):
      pltpu.async_copy(row_h.at[s, b_idx], r, rsem)

    def wait_row(b_idx, r, rsem):
      pltpu.make_async_copy(row_h.at[s, b_idx], r, rsem).wait()

    def issue_gathers(cv, rows, sem):
      for k2 in range(K):
        pltpu.async_copy(ego_h.at[cv.at[pl.ds(k2 * E, E)]],
                         rows.at[pl.ds(k2 * E, E)], sem)

    def drain_gathers(cv, rows, sem):
      for k2 in range(K):
        pltpu.make_async_copy(ego_h.at[cv.at[pl.ds(k2 * E, E)]],
                              rows.at[pl.ds(k2 * E, E)], sem).wait()

    def issue_scatters(rows, r, sem):
      for k2 in range(K):
        pltpu.async_copy(rows.at[pl.ds(k2 * E, E)], acc.at[r.at[k2]],
                         sem, add=True)

    def drain_scatters(rows, r, sem):
      for k2 in range(K):
        pltpu.make_async_copy(rows.at[pl.ds(k2 * E, E)], acc.at[r.at[k2]],
                              sem).wait()

    def multiply(rows, vv):
      def mbody(g, carry):
        vf = vv[pl.ds(g * 16, 16)]
        for j in range(16):
          v = vf[j]
          e = g * 16 + j
          for h2 in range(H // 16):
            rows[e, pl.ds(h2 * 16, 16)] = rows[e, pl.ds(h2 * 16, 16)] * v
        return carry

      lax.fori_loop(0, KE // 16, mbody, 0)

    base0 = s * NCB * KE
    pltpu.sync_copy(col_h.at[pl.ds(base0, KE)], ca)
    pltpu.sync_copy(val_h.at[pl.ds(base0, KE)], va)
    issue_row(0, ra, rsa)
    issue_gathers(ca, rowsa, gsa)

    def body(i, carry):
      a_idx = 2 * i
      b_idx = 2 * i + 1
      pl.when(i > 0)(lambda: drain_scatters(rowsb, rb, ssb))
      issue_cv(b_idx, cb, vb)
      issue_row(b_idx, rb, rsb)
      drain_gathers(ca, rowsa, gsa)
      wait_cv(b_idx, cb, vb)
      issue_gathers(cb, rowsb, gsb)
      multiply(rowsa, va)
      wait_row(a_idx, ra, rsa)
      issue_scatters(rowsa, ra, ssa)
      pl.when(i < NITER - 1)(lambda: issue_cv(2 * i + 2, ca, va))
      drain_gathers(cb, rowsb, gsb)
      multiply(rowsb, vb)
      wait_row(b_idx, rb, rsb)
      issue_scatters(rowsb, rb, ssb)
      drain_scatters(rowsa, ra, ssa)

      def start_a():
        issue_row(2 * i + 2, ra, rsa)
        wait_cv(2 * i + 2, ca, va)
        issue_gathers(ca, rowsa, gsa)

      pl.when(i < NITER - 1)(start_a)
      return carry

    lax.fori_loop(0, NITER, body, 0)
    drain_scatters(rowsb, rb, ssb)
    plsc.subcore_barrier()
    pltpu.sync_copy(acc.at[pl.ds(s * RPT, RPT)],
                    side_h.at[pl.ds(s * RPT, RPT)])

  pl.when(c == 0)(lambda: half(ego_lo, side_lo))
  pl.when(c == 1)(lambda: half(ego_hi, side_hi))


_prop = pl.kernel(
    _prop_body,
    out_type=[jax.ShapeDtypeStruct((NP, H), jnp.float32),
              jax.ShapeDtypeStruct((NP, H), jnp.float32)],
    mesh=plsc.VectorSubcoreMesh(core_axis_name="c", subcore_axis_name="s"),
    compiler_params=pltpu.CompilerParams(use_tc_tiling_on_sc=False,
                                         needs_layout_passes=False),
    scratch_types=[
        pltpu.VMEM_SHARED((NP, H), jnp.float32),
        pltpu.VMEM((K * E,), jnp.int32),
        pltpu.VMEM((K * E,), jnp.int32),
        pltpu.VMEM((K * E,), jnp.float32),
        pltpu.VMEM((K * E,), jnp.float32),
        pltpu.VMEM((K, E), jnp.int32),
        pltpu.VMEM((K, E), jnp.int32),
        pltpu.VMEM((K * E, H), jnp.float32),
        pltpu.VMEM((K * E, H), jnp.float32),
        pltpu.SemaphoreType.DMA,
        pltpu.SemaphoreType.DMA,
        pltpu.SemaphoreType.DMA,
        pltpu.SemaphoreType.DMA,
        pltpu.SemaphoreType.DMA,
        pltpu.SemaphoreType.DMA,
        pltpu.SemaphoreType.DMA,
    ],
)


def _dense_core(slo, shi, elo, ehi, w, b):
  side = jnp.concatenate([slo[...], shi[...]], axis=1)
  eg = jnp.concatenate([elo[...], ehi[...]], axis=1)
  xin = jnp.concatenate([side, eg * side], axis=1)
  t = lax.dot_general(xin, w[...],
                      dimension_numbers=(((1,), (0,)), ((), ())),
                      preferred_element_type=jnp.float32) + b[0:1, :]
  x = jnp.where(t >= 0, t, 0.2 * t)
  nrm = jnp.maximum(jnp.sqrt(jnp.sum(x * x, axis=1, keepdims=True)), 1e-12)
  return x, jnp.concatenate([x / nrm, jnp.zeros_like(x)], axis=1)


def _dense_body(slo, shi, elo, ehi, w, b, nlo, nhi, norm):
  x, n128 = _dense_core(slo, shi, elo, ehi, w, b)
  nlo[...] = x[:, 0:32]
  nhi[...] = x[:, 32:64]
  norm[...] = n128


def _dense_last_body(slo, shi, elo, ehi, w, b, norm):
  _, n128 = _dense_core(slo, shi, elo, ehi, w, b)
  norm[...] = n128


BR = 3128
_row_spec = pl.BlockSpec((BR, H), lambda i: (i, 0))
_w_spec = pl.BlockSpec((2 * D, D), lambda i: (0, 0))
_b_spec = pl.BlockSpec((8, D), lambda i: (0, 0))
_n_spec = pl.BlockSpec((BR, 2 * D), lambda i: (i, 0))


def _dense(slo, shi, elo, ehi, w, b):
  return pl.pallas_call(
      _dense_body,
      grid=(NP // BR,),
      in_specs=[_row_spec, _row_spec, _row_spec, _row_spec, _w_spec, _b_spec],
      out_specs=[_row_spec, _row_spec, _n_spec],
      out_shape=[jax.ShapeDtypeStruct((NP, H), jnp.float32),
                 jax.ShapeDtypeStruct((NP, H), jnp.float32),
                 jax.ShapeDtypeStruct((NP, 2 * D), jnp.float32)],
  )(slo, shi, elo, ehi, w, b)


def _dense_last(slo, shi, elo, ehi, w, b):
  return pl.pallas_call(
      _dense_last_body,
      grid=(NP // BR,),
      in_specs=[_row_spec, _row_spec, _row_spec, _row_spec, _w_spec, _b_spec],
      out_specs=[_n_spec],
      out_shape=[jax.ShapeDtypeStruct((NP, 2 * D), jnp.float32)],
  )(slo, shi, elo, ehi, w, b)


def _gather_body(t0, t1, t2, t3, ru, rp, rn, outu, outp, outn,
                 idxv, gbuf, sem):
  c = lax.axis_index("c")
  s = lax.axis_index("s")
  w = s * NC + c
  tables = [t0, t1, t2, t3]
  for ids_h, out_h in ((ru, outu), (rp, outp), (rn, outn)):
    pltpu.sync_copy(ids_h.at[pl.ds(w * BPW, BPW)], idxv)
    for t in range(4):
      pltpu.async_copy(tables[t].at[idxv], gbuf, sem).wait()
      pltpu.sync_copy(gbuf, out_h.at[t, pl.ds(w * BPW, BPW)])


_gather = pl.kernel(
    _gather_body,
    out_type=[jax.ShapeDtypeStruct((4, B, 2 * D), jnp.float32)] * 3,
    mesh=plsc.VectorSubcoreMesh(core_axis_name="c", subcore_axis_name="s"),
    compiler_params=pltpu.CompilerParams(needs_layout_passes=False),
    scratch_types=[
        pltpu.VMEM((BPW,), jnp.int32),
        pltpu.VMEM((BPW, 2 * D), jnp.float32),
        pltpu.SemaphoreType.DMA,
    ],
)


def _loss_body(u, p, n, loss, bpr, reg):
  ps = 0.0
  ns = 0.0
  for t in range(4):
    ps = ps + jnp.sum(u[t, :, 0:D] * p[t, :, 0:D], axis=1)
    ns = ns + jnp.sum(u[t, :, 0:D] * n[t, :, 0:D], axis=1)
  d = ps - ns
  ls = jnp.minimum(d, 0.0) - jnp.log1p(jnp.exp(-jnp.abs(d)))
  bpr_v = -jnp.sum(ls) / B
  u0 = u[0, :, 0:D]
  p0 = p[0, :, 0:D]
  n0 = n[0, :, 0:D]
  reg_v = DECAY * (jnp.sum(u0 * u0) + jnp.sum(p0 * p0) + jnp.sum(n0 * n0)) \
      / 2.0 / B
  loss[0, 0] = bpr_v + reg_v
  bpr[0, 0] = bpr_v
  reg[0, 0] = reg_v


def _loss(u, p, n):
  spec = pl.BlockSpec((4, B, 2 * D), lambda: (0, 0, 0))
  return pl.pallas_call(
      _loss_body,
      in_specs=[spec, spec, spec],
      out_specs=[pl.BlockSpec(memory_space=pltpu.SMEM)] * 3,
      out_shape=[jax.ShapeDtypeStruct((1, 1), jnp.float32)] * 3,
  )(u, p, n)


def kernel(user_emb, item_emb, W_gc, b_gc, W_bi, b_bi,
           edge_val, edge_row, edge_col, user_id, item_id, neg_item_id):
  ego0 = jnp.concatenate([user_emb, item_emb,
                          jnp.zeros((NP - N, D), jnp.float32)], axis=0)
  elo = ego0[:, :H]
  ehi = ego0[:, H:]

  pad = PADN - NNZ
  pad_rows = (jnp.arange(pad, dtype=jnp.int32) * 64) % N
  col_p = jnp.concatenate([edge_col.astype(jnp.int32), pad_rows])
  row_p = jnp.concatenate([edge_row.astype(jnp.int32), pad_rows])
  val_p = jnp.concatenate([edge_val, jnp.zeros((pad,), jnp.float32)])
  rowp_h = row_p.reshape(NS, NCB, K, E)

  tables = [jnp.concatenate([ego0, jnp.zeros((NP, D), jnp.float32)], axis=1)]
  for k in range(L):
    slo, shi = _prop(elo, ehi, col_p, val_p, rowp_h)
    w = jnp.concatenate([W_gc[k], W_bi[k]], axis=0)
    b = jnp.broadcast_to(b_gc[k] + b_bi[k], (8, D))
    if k < L - 1:
      elo, ehi, nrm = _dense(slo, shi, elo, ehi, w, b)
    else:
      (nrm,) = _dense_last(slo, shi, elo, ehi, w, b)
    tables.append(nrm)

  ru = user_id.astype(jnp.int32)
  rp = item_id.astype(jnp.int32) + USER_NUM
  rn = neg_item_id.astype(jnp.int32) + USER_NUM
  u, p, n = _gather(tables[0], tables[1], tables[2], tables[3], ru, rp, rn)
  loss, bpr, reg = _loss(u, p, n)
  return (loss[0, 0], bpr[0, 0], reg[0, 0])


# speedup vs baseline: 9.8863x; 1.0856x over previous
import functools

import jax
import jax.numpy as jnp
from jax import lax
from jax.experimental import pallas as pl
from jax.experimental.pallas import tpu as pltpu
from jax.experimental.pallas import tpu_sc as plsc

USER_NUM = 25000
ITEM_NUM = 25000
N = USER_NUM + ITEM_NUM
D = 64
H = 32
L = 3
NNZ = 800000
B = 4096
DECAY = 1e-4

NC = 2
NS = 16
NP = 50048
E = 128
K = 3
NCB = 132
NITER = NCB // 2
EPT = E * K * NCB
PADN = EPT * NS
RPT = NP // NS
BPW = B // (NC * NS)


def _prop_body(ego_lo, ego_hi, col_h, val_h, row_h,
               side_lo, side_hi, acc, ca, cb, va, vb, ra, rb, rowsa, rowsb,
               isem, rsa, rsb, gsa, gsb, ssa, ssb):
  c = lax.axis_index("c")
  s = lax.axis_index("s")
  KE = K * E

  def half(ego_h, side_h):
    zero = jnp.zeros((16,), jnp.float32)

    def zbody(r0, carry):
      for h2 in range(H // 16):
        rowsa[r0, pl.ds(h2 * 16, 16)] = zero
      return carry

    lax.fori_loop(0, KE, zbody, 0)
    for i in range(RPT // KE):
      pltpu.sync_copy(rowsa, acc.at[pl.ds(s * RPT + i * KE, KE)])
    rem = RPT - (RPT // KE) * KE
    if rem:
      pltpu.sync_copy(rowsa.at[pl.ds(0, rem)],
                      acc.at[pl.ds(s * RPT + (RPT // KE) * KE, rem)])
    plsc.subcore_barrier()

    def issue_cv(b_idx, cv, vv):
      base = (s * NCB + b_idx) * KE
      pltpu.async_copy(col_h.at[pl.ds(base, KE)], cv, isem)
      pltpu.async_copy(val_h.at[pl.ds(base, KE)], vv, isem)

    def wait_cv(b_idx, cv, vv):
      base = (s * NCB + b_idx) * KE
      pltpu.make_async_copy(col_h.at[pl.ds(base, KE)], cv, isem).wait()
      pltpu.make_async_copy(val_h.at[pl.ds(base, KE)], vv, isem).wait()

    def issue_row(b_idx, r, rsem):
      pltpu.async_copy(row_h.at[s, b_idx], r, rsem)

    def wait_row(b_idx, r, rsem):
      pltpu.make_async_copy(row_h.at[s, b_idx], r, rsem).wait()

    def issue_gathers(cv, rows, sem):
      for k2 in range(K):
        pltpu.async_copy(ego_h.at[cv.at[pl.ds(k2 * E, E)]],
                         rows.at[pl.ds(k2 * E, E)], sem)

    def drain_gathers(cv, rows, sem):
      for k2 in range(K):
        pltpu.make_async_copy(ego_h.at[cv.at[pl.ds(k2 * E, E)]],
                              rows.at[pl.ds(k2 * E, E)], sem).wait()

    def issue_scatters(rows, r, sem):
      for k2 in range(K):
        pltpu.async_copy(rows.at[pl.ds(k2 * E, E)], acc.at[r.at[k2]],
                         sem, add=True)

    def drain_scatters(rows, r, sem):
      for k2 in range(K):
        pltpu.make_async_copy(rows.at[pl.ds(k2 * E, E)], acc.at[r.at[k2]],
                              sem).wait()

    def mult_chunk(rows, vv, k2):
      def mbody(g, carry):
        vf = vv[pl.ds(g * 16, 16)]
        for j in range(16):
          v = vf[j]
          e = g * 16 + j
          for h2 in range(H // 16):
            rows[e, pl.ds(h2 * 16, 16)] = rows[e, pl.ds(h2 * 16, 16)] * v
        return carry

      lax.fori_loop(k2 * (E // 16), (k2 + 1) * (E // 16), mbody, 0)

    def process(cv, vv, rows, r, gsem, ssem):
      for k2 in range(K):
        pltpu.make_async_copy(ego_h.at[cv.at[pl.ds(k2 * E, E)]],
                              rows.at[pl.ds(k2 * E, E)], gsem).wait()
        mult_chunk(rows, vv, k2)
        pltpu.async_copy(rows.at[pl.ds(k2 * E, E)], acc.at[r.at[k2]],
                         ssem, add=True)

    base0 = s * NCB * KE
    pltpu.sync_copy(col_h.at[pl.ds(base0, KE)], ca)
    pltpu.sync_copy(val_h.at[pl.ds(base0, KE)], va)
    issue_row(0, ra, rsa)
    issue_gathers(ca, rowsa, gsa)
    issue_cv(1, cb, vb)

    def body(i, carry):
      a_idx = 2 * i
      b_idx = 2 * i + 1
      pl.when(i > 0)(lambda: drain_scatters(rowsb, rb, ssb))
      issue_row(b_idx, rb, rsb)
      wait_cv(b_idx, cb, vb)
      issue_gathers(cb, rowsb, gsb)
      wait_row(a_idx, ra, rsa)
      process(ca, va, rowsa, ra, gsa, ssa)
      pl.when(i < NITER - 1)(lambda: issue_cv(2 * i + 2, ca, va))
      wait_row(b_idx, rb, rsb)
      process(cb, vb, rowsb, rb, gsb, ssb)
      drain_scatters(rowsa, ra, ssa)

      def start_a():
        issue_row(2 * i + 2, ra, rsa)
        wait_cv(2 * i + 2, ca, va)
        issue_gathers(ca, rowsa, gsa)
        issue_cv(2 * i + 3, cb, vb)

      pl.when(i < NITER - 1)(start_a)
      return carry

    lax.fori_loop(0, NITER, body, 0)
    drain_scatters(rowsb, rb, ssb)
    plsc.subcore_barrier()
    pltpu.sync_copy(acc.at[pl.ds(s * RPT, RPT)],
                    side_h.at[pl.ds(s * RPT, RPT)])

  pl.when(c == 0)(lambda: half(ego_lo, side_lo))
  pl.when(c == 1)(lambda: half(ego_hi, side_hi))


_prop = pl.kernel(
    _prop_body,
    out_type=[jax.ShapeDtypeStruct((NP, H), jnp.float32),
              jax.ShapeDtypeStruct((NP, H), jnp.float32)],
    mesh=plsc.VectorSubcoreMesh(core_axis_name="c", subcore_axis_name="s"),
    compiler_params=pltpu.CompilerParams(use_tc_tiling_on_sc=False,
                                         needs_layout_passes=False),
    scratch_types=[
        pltpu.VMEM_SHARED((NP, H), jnp.float32),
        pltpu.VMEM((K * E,), jnp.int32),
        pltpu.VMEM((K * E,), jnp.int32),
        pltpu.VMEM((K * E,), jnp.float32),
        pltpu.VMEM((K * E,), jnp.float32),
        pltpu.VMEM((K, E), jnp.int32),
        pltpu.VMEM((K, E), jnp.int32),
        pltpu.VMEM((K * E, H), jnp.float32),
        pltpu.VMEM((K * E, H), jnp.float32),
        pltpu.SemaphoreType.DMA,
        pltpu.SemaphoreType.DMA,
        pltpu.SemaphoreType.DMA,
        pltpu.SemaphoreType.DMA,
        pltpu.SemaphoreType.DMA,
        pltpu.SemaphoreType.DMA,
        pltpu.SemaphoreType.DMA,
    ],
)


def _dense_core(slo, shi, elo, ehi, w, b):
  side = jnp.concatenate([slo[...], shi[...]], axis=1)
  eg = jnp.concatenate([elo[...], ehi[...]], axis=1)
  xin = jnp.concatenate([side, eg * side], axis=1)
  t = lax.dot_general(xin, w[...],
                      dimension_numbers=(((1,), (0,)), ((), ())),
                      preferred_element_type=jnp.float32) + b[0:1, :]
  x = jnp.where(t >= 0, t, 0.2 * t)
  nrm = jnp.maximum(jnp.sqrt(jnp.sum(x * x, axis=1, keepdims=True)), 1e-12)
  return x, jnp.concatenate([x / nrm, jnp.zeros_like(x)], axis=1)


def _dense_body(slo, shi, elo, ehi, w, b, nlo, nhi, norm):
  x, n128 = _dense_core(slo, shi, elo, ehi, w, b)
  nlo[...] = x[:, 0:32]
  nhi[...] = x[:, 32:64]
  norm[...] = n128


def _dense_last_body(slo, shi, elo, ehi, w, b, norm):
  _, n128 = _dense_core(slo, shi, elo, ehi, w, b)
  norm[...] = n128


BR = 3128
_row_spec = pl.BlockSpec((BR, H), lambda i: (i, 0))
_w_spec = pl.BlockSpec((2 * D, D), lambda i: (0, 0))
_b_spec = pl.BlockSpec((8, D), lambda i: (0, 0))
_n_spec = pl.BlockSpec((BR, 2 * D), lambda i: (i, 0))


def _dense(slo, shi, elo, ehi, w, b):
  return pl.pallas_call(
      _dense_body,
      grid=(NP // BR,),
      in_specs=[_row_spec, _row_spec, _row_spec, _row_spec, _w_spec, _b_spec],
      out_specs=[_row_spec, _row_spec, _n_spec],
      out_shape=[jax.ShapeDtypeStruct((NP, H), jnp.float32),
                 jax.ShapeDtypeStruct((NP, H), jnp.float32),
                 jax.ShapeDtypeStruct((NP, 2 * D), jnp.float32)],
  )(slo, shi, elo, ehi, w, b)


def _dense_last(slo, shi, elo, ehi, w, b):
  return pl.pallas_call(
      _dense_last_body,
      grid=(NP // BR,),
      in_specs=[_row_spec, _row_spec, _row_spec, _row_spec, _w_spec, _b_spec],
      out_specs=[_n_spec],
      out_shape=[jax.ShapeDtypeStruct((NP, 2 * D), jnp.float32)],
  )(slo, shi, elo, ehi, w, b)


def _gather_body(t0, t1, t2, t3, ru, rp, rn, outu, outp, outn,
                 idxv, gbuf, sem):
  c = lax.axis_index("c")
  s = lax.axis_index("s")
  w = s * NC + c
  tables = [t0, t1, t2, t3]
  for ids_h, out_h in ((ru, outu), (rp, outp), (rn, outn)):
    pltpu.sync_copy(ids_h.at[pl.ds(w * BPW, BPW)], idxv)
    for t in range(4):
      pltpu.async_copy(tables[t].at[idxv], gbuf, sem).wait()
      pltpu.sync_copy(gbuf, out_h.at[t, pl.ds(w * BPW, BPW)])


_gather = pl.kernel(
    _gather_body,
    out_type=[jax.ShapeDtypeStruct((4, B, 2 * D), jnp.float32)] * 3,
    mesh=plsc.VectorSubcoreMesh(core_axis_name="c", subcore_axis_name="s"),
    compiler_params=pltpu.CompilerParams(needs_layout_passes=False),
    scratch_types=[
        pltpu.VMEM((BPW,), jnp.int32),
        pltpu.VMEM((BPW, 2 * D), jnp.float32),
        pltpu.SemaphoreType.DMA,
    ],
)


def _loss_body(u, p, n, loss, bpr, reg):
  ps = 0.0
  ns = 0.0
  for t in range(4):
    ps = ps + jnp.sum(u[t, :, 0:D] * p[t, :, 0:D], axis=1)
    ns = ns + jnp.sum(u[t, :, 0:D] * n[t, :, 0:D], axis=1)
  d = ps - ns
  ls = jnp.minimum(d, 0.0) - jnp.log1p(jnp.exp(-jnp.abs(d)))
  bpr_v = -jnp.sum(ls) / B
  u0 = u[0, :, 0:D]
  p0 = p[0, :, 0:D]
  n0 = n[0, :, 0:D]
  reg_v = DECAY * (jnp.sum(u0 * u0) + jnp.sum(p0 * p0) + jnp.sum(n0 * n0)) \
      / 2.0 / B
  loss[0, 0] = bpr_v + reg_v
  bpr[0, 0] = bpr_v
  reg[0, 0] = reg_v


def _loss(u, p, n):
  spec = pl.BlockSpec((4, B, 2 * D), lambda: (0, 0, 0))
  return pl.pallas_call(
      _loss_body,
      in_specs=[spec, spec, spec],
      out_specs=[pl.BlockSpec(memory_space=pltpu.SMEM)] * 3,
      out_shape=[jax.ShapeDtypeStruct((1, 1), jnp.float32)] * 3,
  )(u, p, n)


def kernel(user_emb, item_emb, W_gc, b_gc, W_bi, b_bi,
           edge_val, edge_row, edge_col, user_id, item_id, neg_item_id):
  ego0 = jnp.concatenate([user_emb, item_emb,
                          jnp.zeros((NP - N, D), jnp.float32)], axis=0)
  elo = ego0[:, :H]
  ehi = ego0[:, H:]

  pad = PADN - NNZ
  pad_rows = (jnp.arange(pad, dtype=jnp.int32) * 64) % N
  col_p = jnp.concatenate([edge_col.astype(jnp.int32), pad_rows])
  row_p = jnp.concatenate([edge_row.astype(jnp.int32), pad_rows])
  val_p = jnp.concatenate([edge_val, jnp.zeros((pad,), jnp.float32)])
  rowp_h = row_p.reshape(NS, NCB, K, E)

  tables = [jnp.concatenate([ego0, jnp.zeros((NP, D), jnp.float32)], axis=1)]
  for k in range(L):
    slo, shi = _prop(elo, ehi, col_p, val_p, rowp_h)
    w = jnp.concatenate([W_gc[k], W_bi[k]], axis=0)
    b = jnp.broadcast_to(b_gc[k] + b_bi[k], (8, D))
    if k < L - 1:
      elo, ehi, nrm = _dense(slo, shi, elo, ehi, w, b)
    else:
      (nrm,) = _dense_last(slo, shi, elo, ehi, w, b)
    tables.append(nrm)

  ru = user_id.astype(jnp.int32)
  rp = item_id.astype(jnp.int32) + USER_NUM
  rn = neg_item_id.astype(jnp.int32) + USER_NUM
  u, p, n = _gather(tables[0], tables[1], tables[2], tables[3], ru, rp, rn)
  loss, bpr, reg = _loss(u, p, n)
  return (loss[0, 0], bpr[0, 0], reg[0, 0])


# speedup vs baseline: 9.9184x; 1.0032x over previous
import jax
import jax.numpy as jnp
from jax import lax
from jax.experimental import pallas as pl
from jax.experimental.pallas import tpu as pltpu
from jax.experimental.pallas import tpu_sc as plsc

USER_NUM = 25000
ITEM_NUM = 25000
N = USER_NUM + ITEM_NUM
D = 64
H = 32
L = 3
NNZ = 800000
B = 4096
DECAY = 1e-4

NC = 2
NS = 16
NP = 50048
E = 128
K = 3
NCB = 132
NITER = NCB // 2
EPT = E * K * NCB
PADN = EPT * NS
RPT = NP // NS
BPW = B // (NC * NS)


def _prop_body(ego_lo, ego_hi, col_h, val_h, row_h,
               side_lo, side_hi, acc, ca, cb, va, vb, ra, rb, rowsa, rowsb,
               isem, rsa, rsb, gsa, gsb, ssa, ssb):
  c = lax.axis_index("c")
  s = lax.axis_index("s")
  KE = K * E

  def half(ego_h, side_h):
    zero = jnp.zeros((16,), jnp.float32)

    def zbody(r0, carry):
      for h2 in range(H // 16):
        rowsa[r0, pl.ds(h2 * 16, 16)] = zero
      return carry

    lax.fori_loop(0, KE, zbody, 0)
    for i in range(RPT // KE):
      pltpu.sync_copy(rowsa, acc.at[pl.ds(s * RPT + i * KE, KE)])
    rem = RPT - (RPT // KE) * KE
    if rem:
      pltpu.sync_copy(rowsa.at[pl.ds(0, rem)],
                      acc.at[pl.ds(s * RPT + (RPT // KE) * KE, rem)])
    plsc.subcore_barrier()

    def issue_cv(b_idx, cv, vv):
      base = (s * NCB + b_idx) * KE
      pltpu.async_copy(col_h.at[pl.ds(base, KE)], cv, isem)
      pltpu.async_copy(val_h.at[pl.ds(base, KE)], vv, isem)

    def wait_cv(b_idx, cv, vv):
      base = (s * NCB + b_idx) * KE
      pltpu.make_async_copy(col_h.at[pl.ds(base, KE)], cv, isem).wait()
      pltpu.make_async_copy(val_h.at[pl.ds(base, KE)], vv, isem).wait()

    def issue_row(b_idx, r, rsem):
      pltpu.async_copy(row_h.at[s, b_idx], r, rsem)

    def wait_row(b_idx, r, rsem):
      pltpu.make_async_copy(row_h.at[s, b_idx], r, rsem).wait()

    def issue_gathers(cv, rows, sem):
      for k2 in range(K):
        pltpu.async_copy(ego_h.at[cv.at[pl.ds(k2 * E, E)]],
                         rows.at[pl.ds(k2 * E, E)], sem)

    def drain_gathers(cv, rows, sem):
      for k2 in range(K):
        pltpu.make_async_copy(ego_h.at[cv.at[pl.ds(k2 * E, E)]],
                              rows.at[pl.ds(k2 * E, E)], sem).wait()

    def issue_scatters(rows, r, sem):
      for k2 in range(K):
        pltpu.async_copy(rows.at[pl.ds(k2 * E, E)], acc.at[r.at[k2]],
                         sem, add=True)

    def drain_scatters(rows, r, sem):
      for k2 in range(K):
        pltpu.make_async_copy(rows.at[pl.ds(k2 * E, E)], acc.at[r.at[k2]],
                              sem).wait()

    def mult_chunk(rows, vv, k2):
      def mbody(g, carry):
        vf = vv[pl.ds(g * 16, 16)]
        for j in range(16):
          v = vf[j]
          e = g * 16 + j
          for h2 in range(H // 16):
            rows[e, pl.ds(h2 * 16, 16)] = rows[e, pl.ds(h2 * 16, 16)] * v
        return carry

      lax.fori_loop(k2 * (E // 16), (k2 + 1) * (E // 16), mbody, 0)

    def process(cv, vv, rows, r, gsem, ssem):
      for k2 in range(K):
        pltpu.make_async_copy(ego_h.at[cv.at[pl.ds(k2 * E, E)]],
                              rows.at[pl.ds(k2 * E, E)], gsem).wait()
        mult_chunk(rows, vv, k2)
        pltpu.async_copy(rows.at[pl.ds(k2 * E, E)], acc.at[r.at[k2]],
                         ssem, add=True)

    base0 = s * NCB * KE
    pltpu.sync_copy(col_h.at[pl.ds(base0, KE)], ca)
    pltpu.sync_copy(val_h.at[pl.ds(base0, KE)], va)
    issue_row(0, ra, rsa)
    issue_gathers(ca, rowsa, gsa)
    issue_cv(1, cb, vb)

    def body(i, carry):
      a_idx = 2 * i
      b_idx = 2 * i + 1
      pl.when(i > 0)(lambda: drain_scatters(rowsb, rb, ssb))
      issue_row(b_idx, rb, rsb)
      wait_cv(b_idx, cb, vb)
      issue_gathers(cb, rowsb, gsb)
      wait_row(a_idx, ra, rsa)
      process(ca, va, rowsa, ra, gsa, ssa)
      pl.when(i < NITER - 1)(lambda: issue_cv(2 * i + 2, ca, va))
      wait_row(b_idx, rb, rsb)
      process(cb, vb, rowsb, rb, gsb, ssb)
      drain_scatters(rowsa, ra, ssa)

      def start_a():
        issue_row(2 * i + 2, ra, rsa)
        wait_cv(2 * i + 2, ca, va)
        issue_gathers(ca, rowsa, gsa)
        issue_cv(2 * i + 3, cb, vb)

      pl.when(i < NITER - 1)(start_a)
      return carry

    lax.fori_loop(0, NITER, body, 0)
    drain_scatters(rowsb, rb, ssb)
    plsc.subcore_barrier()
    pltpu.sync_copy(acc.at[pl.ds(s * RPT, RPT)],
                    side_h.at[pl.ds(s * RPT, RPT)])

  pl.when(c == 0)(lambda: half(ego_lo, side_lo))
  pl.when(c == 1)(lambda: half(ego_hi, side_hi))


_prop = pl.kernel(
    _prop_body,
    out_type=[jax.ShapeDtypeStruct((NP, H), jnp.float32),
              jax.ShapeDtypeStruct((NP, H), jnp.float32)],
    mesh=plsc.VectorSubcoreMesh(core_axis_name="c", subcore_axis_name="s"),
    compiler_params=pltpu.CompilerParams(use_tc_tiling_on_sc=False,
                                         needs_layout_passes=False),
    scratch_types=[
        pltpu.VMEM_SHARED((NP, H), jnp.float32),
        pltpu.VMEM((K * E,), jnp.int32),
        pltpu.VMEM((K * E,), jnp.int32),
        pltpu.VMEM((K * E,), jnp.float32),
        pltpu.VMEM((K * E,), jnp.float32),
        pltpu.VMEM((K, E), jnp.int32),
        pltpu.VMEM((K, E), jnp.int32),
        pltpu.VMEM((K * E, H), jnp.float32),
        pltpu.VMEM((K * E, H), jnp.float32),
        pltpu.SemaphoreType.DMA,
        pltpu.SemaphoreType.DMA,
        pltpu.SemaphoreType.DMA,
        pltpu.SemaphoreType.DMA,
        pltpu.SemaphoreType.DMA,
        pltpu.SemaphoreType.DMA,
        pltpu.SemaphoreType.DMA,
    ],
)


def _dense_core(slo, shi, elo, ehi, w, b):
  side = jnp.concatenate([slo[...], shi[...]], axis=1)
  eg = jnp.concatenate([elo[...], ehi[...]], axis=1)
  xin = jnp.concatenate([side, eg * side], axis=1)
  t = lax.dot_general(xin, w[...],
                      dimension_numbers=(((1,), (0,)), ((), ())),
                      preferred_element_type=jnp.float32) + b[0:1, :]
  x = jnp.where(t >= 0, t, 0.2 * t)
  nrm = jnp.maximum(jnp.sqrt(jnp.sum(x * x, axis=1, keepdims=True)), 1e-12)
  return x, jnp.concatenate([x / nrm, jnp.zeros_like(x)], axis=1)


def _dense_body(slo, shi, elo, ehi, w, b, nlo, nhi, norm):
  x, n128 = _dense_core(slo, shi, elo, ehi, w, b)
  nlo[...] = x[:, 0:32]
  nhi[...] = x[:, 32:64]
  norm[...] = n128


def _dense_last_body(slo, shi, elo, ehi, w, b, norm):
  _, n128 = _dense_core(slo, shi, elo, ehi, w, b)
  norm[...] = n128


BR = 6256
_row_spec = pl.BlockSpec((BR, H), lambda i: (i, 0))
_w_spec = pl.BlockSpec((2 * D, D), lambda i: (0, 0))
_b_spec = pl.BlockSpec((8, D), lambda i: (0, 0))
_n_spec = pl.BlockSpec((BR, 2 * D), lambda i: (i, 0))


def _dense(slo, shi, elo, ehi, w, b):
  return pl.pallas_call(
      _dense_body,
      grid=(NP // BR,),
      in_specs=[_row_spec, _row_spec, _row_spec, _row_spec, _w_spec, _b_spec],
      out_specs=[_row_spec, _row_spec, _n_spec],
      out_shape=[jax.ShapeDtypeStruct((NP, H), jnp.float32),
                 jax.ShapeDtypeStruct((NP, H), jnp.float32),
                 jax.ShapeDtypeStruct((NP, 2 * D), jnp.float32)],
  )(slo, shi, elo, ehi, w, b)


def _dense_last(slo, shi, elo, ehi, w, b):
  return pl.pallas_call(
      _dense_last_body,
      grid=(NP // BR,),
      in_specs=[_row_spec, _row_spec, _row_spec, _row_spec, _w_spec, _b_spec],
      out_specs=[_n_spec],
      out_shape=[jax.ShapeDtypeStruct((NP, 2 * D), jnp.float32)],
  )(slo, shi, elo, ehi, w, b)


def _gather_body(t0, t1, t2, t3, ru, rp, rn, outu, outp, outn,
                 idxv, gbuf, sem):
  c = lax.axis_index("c")
  s = lax.axis_index("s")
  w = s * NC + c
  tables = [t0, t1, t2, t3]
  for ids_h, out_h in ((ru, outu), (rp, outp), (rn, outn)):
    pltpu.sync_copy(ids_h.at[pl.ds(w * BPW, BPW)], idxv)
    for t in range(4):
      pltpu.async_copy(tables[t].at[idxv], gbuf, sem).wait()
      pltpu.sync_copy(gbuf, out_h.at[t, pl.ds(w * BPW, BPW)])


_gather = pl.kernel(
    _gather_body,
    out_type=[jax.ShapeDtypeStruct((4, B, 2 * D), jnp.float32)] * 3,
    mesh=plsc.VectorSubcoreMesh(core_axis_name="c", subcore_axis_name="s"),
    compiler_params=pltpu.CompilerParams(needs_layout_passes=False),
    scratch_types=[
        pltpu.VMEM((BPW,), jnp.int32),
        pltpu.VMEM((BPW, 2 * D), jnp.float32),
        pltpu.SemaphoreType.DMA,
    ],
)


def _loss_body(u, p, n, loss, bpr, reg):
  ps = 0.0
  ns = 0.0
  for t in range(4):
    ps = ps + jnp.sum(u[t, :, 0:D] * p[t, :, 0:D], axis=1)
    ns = ns + jnp.sum(u[t, :, 0:D] * n[t, :, 0:D], axis=1)
  d = ps - ns
  ls = jnp.minimum(d, 0.0) - jnp.log1p(jnp.exp(-jnp.abs(d)))
  bpr_v = -jnp.sum(ls) / B
  u0 = u[0, :, 0:D]
  p0 = p[0, :, 0:D]
  n0 = n[0, :, 0:D]
  reg_v = DECAY * (jnp.sum(u0 * u0) + jnp.sum(p0 * p0) + jnp.sum(n0 * n0)) \
      / 2.0 / B
  loss[0, 0] = bpr_v + reg_v
  bpr[0, 0] = bpr_v
  reg[0, 0] = reg_v


def _loss(u, p, n):
  spec = pl.BlockSpec((4, B, 2 * D), lambda: (0, 0, 0))
  return pl.pallas_call(
      _loss_body,
      in_specs=[spec, spec, spec],
      out_specs=[pl.BlockSpec(memory_space=pltpu.SMEM)] * 3,
      out_shape=[jax.ShapeDtypeStruct((1, 1), jnp.float32)] * 3,
  )(u, p, n)


def kernel(user_emb, item_emb, W_gc, b_gc, W_bi, b_bi,
           edge_val, edge_row, edge_col, user_id, item_id, neg_item_id):
  ego0 = jnp.concatenate([user_emb, item_emb,
                          jnp.zeros((NP - N, D), jnp.float32)], axis=0)
  elo = ego0[:, :H]
  ehi = ego0[:, H:]

  pad = PADN - NNZ
  pad_rows = (jnp.arange(pad, dtype=jnp.int32) * 64) % N
  col_p = jnp.concatenate([edge_col.astype(jnp.int32), pad_rows])
  row_p = jnp.concatenate([edge_row.astype(jnp.int32), pad_rows])
  val_p = jnp.concatenate([edge_val, jnp.zeros((pad,), jnp.float32)])
  rowp_h = row_p.reshape(NS, NCB, K, E)

  tables = [jnp.concatenate([ego0, jnp.zeros((NP, D), jnp.float32)], axis=1)]
  for k in range(L):
    slo, shi = _prop(elo, ehi, col_p, val_p, rowp_h)
    w = jnp.concatenate([W_gc[k], W_bi[k]], axis=0)
    b = jnp.broadcast_to(b_gc[k] + b_bi[k], (8, D))
    if k < L - 1:
      elo, ehi, nrm = _dense(slo, shi, elo, ehi, w, b)
    else:
      (nrm,) = _dense_last(slo, shi, elo, ehi, w, b)
    tables.append(nrm)

  ru = user_id.astype(jnp.int32)
  rp = item_id.astype(jnp.int32) + USER_NUM
  rn = neg_item_id.astype(jnp.int32) + USER_NUM
  u, p, n = _gather(tables[0], tables[1], tables[2], tables[3], ru, rp, rn)
  loss, bpr, reg = _loss(u, p, n)
  return (loss[0, 0], bpr[0, 0], reg[0, 0])


# speedup vs baseline: 9.9234x; 1.0005x over previous
import jax
import jax.numpy as jnp
from jax import lax
from jax.experimental import pallas as pl
from jax.experimental.pallas import tpu as pltpu
from jax.experimental.pallas import tpu_sc as plsc

USER_NUM = 25000
ITEM_NUM = 25000
N = USER_NUM + ITEM_NUM
D = 64
H = 32
L = 3
NNZ = 800000
B = 4096
DECAY = 1e-4

NC = 2
NS = 16
NP = 50048
E = 128
K = 3
NCB = 132
NITER = NCB // 2
EPT = E * K * NCB
PADN = EPT * NS
RPT = NP // NS
BPW = B // (NC * NS)


def _prop_body(ego_lo, ego_hi, col_h, val_h, row_h,
               side_lo, side_hi, acc, ca, cb, va, vb, ra, rb, rowsa, rowsb,
               isem, rsa, rsb, gsa, gsb, ssa, ssb):
  c = lax.axis_index("c")
  s = lax.axis_index("s")
  KE = K * E

  def half(ego_h, side_h):
    zero = jnp.zeros((16,), jnp.float32)

    def zbody(r0, carry):
      for h2 in range(H // 16):
        rowsa[r0, pl.ds(h2 * 16, 16)] = zero
      return carry

    lax.fori_loop(0, KE, zbody, 0)
    for i in range(RPT // KE):
      pltpu.sync_copy(rowsa, acc.at[pl.ds(s * RPT + i * KE, KE)])
    rem = RPT - (RPT // KE) * KE
    if rem:
      pltpu.sync_copy(rowsa.at[pl.ds(0, rem)],
                      acc.at[pl.ds(s * RPT + (RPT // KE) * KE, rem)])
    plsc.subcore_barrier()

    def issue_cv(b_idx, cv, vv):
      base = (s * NCB + b_idx) * KE
      pltpu.async_copy(col_h.at[pl.ds(base, KE)], cv, isem)
      pltpu.async_copy(val_h.at[pl.ds(base, KE)], vv, isem)

    def wait_cv(b_idx, cv, vv):
      base = (s * NCB + b_idx) * KE
      pltpu.make_async_copy(col_h.at[pl.ds(base, KE)], cv, isem).wait()
      pltpu.make_async_copy(val_h.at[pl.ds(base, KE)], vv, isem).wait()

    def issue_row(b_idx, r, rsem):
      pltpu.async_copy(row_h.at[s, b_idx], r, rsem)

    def wait_row(b_idx, r, rsem):
      pltpu.make_async_copy(row_h.at[s, b_idx], r, rsem).wait()

    def issue_gathers(cv, rows, sem):
      for k2 in range(K):
        pltpu.async_copy(ego_h.at[cv.at[pl.ds(k2 * E, E)]],
                         rows.at[pl.ds(k2 * E, E)], sem)

    def drain_gathers(cv, rows, sem):
      for k2 in range(K):
        pltpu.make_async_copy(ego_h.at[cv.at[pl.ds(k2 * E, E)]],
                              rows.at[pl.ds(k2 * E, E)], sem).wait()

    def issue_scatters(rows, r, sem):
      for k2 in range(K):
        pltpu.async_copy(rows.at[pl.ds(k2 * E, E)], acc.at[r.at[k2]],
                         sem, add=True)

    def drain_scatters(rows, r, sem):
      for k2 in range(K):
        pltpu.make_async_copy(rows.at[pl.ds(k2 * E, E)], acc.at[r.at[k2]],
                              sem).wait()

    def mult_chunk(rows, vv, k2):
      def mbody(g, carry):
        vf = vv[pl.ds(g * 16, 16)]
        for j in range(16):
          v = vf[j]
          e = g * 16 + j
          for h2 in range(H // 16):
            rows[e, pl.ds(h2 * 16, 16)] = rows[e, pl.ds(h2 * 16, 16)] * v
        return carry

      lax.fori_loop(k2 * (E // 16), (k2 + 1) * (E // 16), mbody, 0)

    def process(cv, vv, rows, r, gsem, ssem):
      for k2 in range(K):
        pltpu.make_async_copy(ego_h.at[cv.at[pl.ds(k2 * E, E)]],
                              rows.at[pl.ds(k2 * E, E)], gsem).wait()
        mult_chunk(rows, vv, k2)
        pltpu.async_copy(rows.at[pl.ds(k2 * E, E)], acc.at[r.at[k2]],
                         ssem, add=True)

    base0 = s * NCB * KE
    pltpu.sync_copy(col_h.at[pl.ds(base0, KE)], ca)
    pltpu.sync_copy(val_h.at[pl.ds(base0, KE)], va)
    issue_row(0, ra, rsa)
    issue_gathers(ca, rowsa, gsa)
    issue_cv(1, cb, vb)

    def body(i, carry):
      a_idx = 2 * i
      b_idx = 2 * i + 1
      pl.when(i > 0)(lambda: drain_scatters(rowsb, rb, ssb))
      issue_row(b_idx, rb, rsb)
      wait_cv(b_idx, cb, vb)
      issue_gathers(cb, rowsb, gsb)
      wait_row(a_idx, ra, rsa)
      process(ca, va, rowsa, ra, gsa, ssa)
      pl.when(i < NITER - 1)(lambda: issue_cv(2 * i + 2, ca, va))
      wait_row(b_idx, rb, rsb)
      process(cb, vb, rowsb, rb, gsb, ssb)
      drain_scatters(rowsa, ra, ssa)

      def start_a():
        issue_row(2 * i + 2, ra, rsa)
        wait_cv(2 * i + 2, ca, va)
        issue_gathers(ca, rowsa, gsa)
        issue_cv(2 * i + 3, cb, vb)

      pl.when(i < NITER - 1)(start_a)
      return carry

    lax.fori_loop(0, NITER, body, 0)
    drain_scatters(rowsb, rb, ssb)
    plsc.subcore_barrier()
    pltpu.sync_copy(acc.at[pl.ds(s * RPT, RPT)],
                    side_h.at[pl.ds(s * RPT, RPT)])

  pl.when(c == 0)(lambda: half(ego_lo, side_lo))
  pl.when(c == 1)(lambda: half(ego_hi, side_hi))


_prop = pl.kernel(
    _prop_body,
    out_type=[jax.ShapeDtypeStruct((NP, H), jnp.float32),
              jax.ShapeDtypeStruct((NP, H), jnp.float32)],
    mesh=plsc.VectorSubcoreMesh(core_axis_name="c", subcore_axis_name="s",
                                num_cores=NC, num_subcores=NS),
    compiler_params=pltpu.CompilerParams(use_tc_tiling_on_sc=False,
                                         needs_layout_passes=False),
    scratch_types=[
        pltpu.VMEM_SHARED((NP, H), jnp.float32),
        pltpu.VMEM((K * E,), jnp.int32),
        pltpu.VMEM((K * E,), jnp.int32),
        pltpu.VMEM((K * E,), jnp.float32),
        pltpu.VMEM((K * E,), jnp.float32),
        pltpu.VMEM((K, E), jnp.int32),
        pltpu.VMEM((K, E), jnp.int32),
        pltpu.VMEM((K * E, H), jnp.float32),
        pltpu.VMEM((K * E, H), jnp.float32),
        pltpu.SemaphoreType.DMA,
        pltpu.SemaphoreType.DMA,
        pltpu.SemaphoreType.DMA,
        pltpu.SemaphoreType.DMA,
        pltpu.SemaphoreType.DMA,
        pltpu.SemaphoreType.DMA,
        pltpu.SemaphoreType.DMA,
    ],
)


def _dense_core(slo, shi, elo, ehi, w, b):
  side = jnp.concatenate([slo[...], shi[...]], axis=1)
  eg = jnp.concatenate([elo[...], ehi[...]], axis=1)
  xin = jnp.concatenate([side, eg * side], axis=1)
  t = lax.dot_general(xin, w[...],
                      dimension_numbers=(((1,), (0,)), ((), ())),
                      preferred_element_type=jnp.float32) + b[0:1, :]
  x = jnp.where(t >= 0, t, 0.2 * t)
  nrm = jnp.maximum(jnp.sqrt(jnp.sum(x * x, axis=1, keepdims=True)), 1e-12)
  return x, jnp.concatenate([x / nrm, jnp.zeros_like(x)], axis=1)


def _dense_body(slo, shi, elo, ehi, w, b, nlo, nhi, norm):
  x, n128 = _dense_core(slo, shi, elo, ehi, w, b)
  nlo[...] = x[:, 0:32]
  nhi[...] = x[:, 32:64]
  norm[...] = n128


def _dense_last_body(slo, shi, elo, ehi, w, b, norm):
  _, n128 = _dense_core(slo, shi, elo, ehi, w, b)
  norm[...] = n128


BR = 6256
_row_spec = pl.BlockSpec((BR, H), lambda i: (i, 0))
_w_spec = pl.BlockSpec((2 * D, D), lambda i: (0, 0))
_b_spec = pl.BlockSpec((8, D), lambda i: (0, 0))
_n_spec = pl.BlockSpec((BR, 2 * D), lambda i: (i, 0))


def _dense(slo, shi, elo, ehi, w, b):
  return pl.pallas_call(
      _dense_body,
      grid=(NP // BR,),
      in_specs=[_row_spec, _row_spec, _row_spec, _row_spec, _w_spec, _b_spec],
      out_specs=[_row_spec, _row_spec, _n_spec],
      out_shape=[jax.ShapeDtypeStruct((NP, H), jnp.float32),
                 jax.ShapeDtypeStruct((NP, H), jnp.float32),
                 jax.ShapeDtypeStruct((NP, 2 * D), jnp.float32)],
  )(slo, shi, elo, ehi, w, b)


def _dense_last(slo, shi, elo, ehi, w, b):
  return pl.pallas_call(
      _dense_last_body,
      grid=(NP // BR,),
      in_specs=[_row_spec, _row_spec, _row_spec, _row_spec, _w_spec, _b_spec],
      out_specs=[_n_spec],
      out_shape=[jax.ShapeDtypeStruct((NP, 2 * D), jnp.float32)],
  )(slo, shi, elo, ehi, w, b)


def _gather_body(t0, t1, t2, t3, ru, rp, rn, outu, outp, outn,
                 idxv, gbuf, sem):
  c = lax.axis_index("c")
  s = lax.axis_index("s")
  w = s * NC + c
  tables = [t0, t1, t2, t3]
  for ids_h, out_h in ((ru, outu), (rp, outp), (rn, outn)):
    pltpu.sync_copy(ids_h.at[pl.ds(w * BPW, BPW)], idxv)
    for t in range(4):
      pltpu.async_copy(tables[t].at[idxv], gbuf, sem).wait()
      pltpu.sync_copy(gbuf, out_h.at[t, pl.ds(w * BPW, BPW)])


_gather = pl.kernel(
    _gather_body,
    out_type=[jax.ShapeDtypeStruct((4, B, 2 * D), jnp.float32)] * 3,
    mesh=plsc.VectorSubcoreMesh(core_axis_name="c", subcore_axis_name="s",
                                num_cores=NC, num_subcores=NS),
    compiler_params=pltpu.CompilerParams(needs_layout_passes=False),
    scratch_types=[
        pltpu.VMEM((BPW,), jnp.int32),
        pltpu.VMEM((BPW, 2 * D), jnp.float32),
        pltpu.SemaphoreType.DMA,
    ],
)


def _loss_body(u, p, n, loss, bpr, reg):
  ps = 0.0
  ns = 0.0
  for t in range(4):
    ps = ps + jnp.sum(u[t, :, 0:D] * p[t, :, 0:D], axis=1)
    ns = ns + jnp.sum(u[t, :, 0:D] * n[t, :, 0:D], axis=1)
  d = ps - ns
  ls = jnp.minimum(d, 0.0) - jnp.log1p(jnp.exp(-jnp.abs(d)))
  bpr_v = -jnp.sum(ls) / B
  u0 = u[0, :, 0:D]
  p0 = p[0, :, 0:D]
  n0 = n[0, :, 0:D]
  reg_v = DECAY * (jnp.sum(u0 * u0) + jnp.sum(p0 * p0) + jnp.sum(n0 * n0)) \
      / 2.0 / B
  loss[0, 0] = bpr_v + reg_v
  bpr[0, 0] = bpr_v
  reg[0, 0] = reg_v


def _loss(u, p, n):
  spec = pl.BlockSpec((4, B, 2 * D), lambda: (0, 0, 0))
  return pl.pallas_call(
      _loss_body,
      in_specs=[spec, spec, spec],
      out_specs=[pl.BlockSpec(memory_space=pltpu.SMEM)] * 3,
      out_shape=[jax.ShapeDtypeStruct((1, 1), jnp.float32)] * 3,
  )(u, p, n)


def kernel(user_emb, item_emb, W_gc, b_gc, W_bi, b_bi,
           edge_val, edge_row, edge_col, user_id, item_id, neg_item_id):
  ego0 = jnp.concatenate([user_emb, item_emb,
                          jnp.zeros((NP - N, D), jnp.float32)], axis=0)
  elo = ego0[:, :H]
  ehi = ego0[:, H:]

  pad = PADN - NNZ
  pad_rows = (jnp.arange(pad, dtype=jnp.int32) * 64) % N
  col_p = jnp.concatenate([edge_col.astype(jnp.int32), pad_rows])
  row_p = jnp.concatenate([edge_row.astype(jnp.int32), pad_rows])
  val_p = jnp.concatenate([edge_val, jnp.zeros((pad,), jnp.float32)])
  rowp_h = row_p.reshape(NS, NCB, K, E)

  tables = [jnp.concatenate([ego0, jnp.zeros((NP, D), jnp.float32)], axis=1)]
  for k in range(L):
    slo, shi = _prop(elo, ehi, col_p, val_p, rowp_h)
    w = jnp.concatenate([W_gc[k], W_bi[k]], axis=0)
    b = jnp.broadcast_to(b_gc[k] + b_bi[k], (8, D))
    if k < L - 1:
      elo, ehi, nrm = _dense(slo, shi, elo, ehi, w, b)
    else:
      (nrm,) = _dense_last(slo, shi, elo, ehi, w, b)
    tables.append(nrm)

  ru = user_id.astype(jnp.int32)
  rp = item_id.astype(jnp.int32) + USER_NUM
  rn = neg_item_id.astype(jnp.int32) + USER_NUM
  u, p, n = _gather(tables[0], tables[1], tables[2], tables[3], ru, rp, rn)
  loss, bpr, reg = _loss(u, p, n)
  return (loss[0, 0], bpr[0, 0], reg[0, 0])
